# baseline (speedup 1.0000x reference)
import functools
import math

import jax
import jax.numpy as jnp
from jax import lax
from jax.experimental import pallas as pl
from jax.experimental.pallas import tpu as pltpu
from jax.experimental.pallas import tpu_sc as plsc

_B, _T, _N, _E = 2, 8, 1000, 16000
_D = 64
_MODES = 4
_TIME_EMB = 32
_BN = _B * _N
_BTN = _B * _T * _N
_BE = _B * _E
_ET = _T * _BE
_GW = 80
_HW = 64
_XW = 16
_CH = 800
_NW = 32

_f32 = jnp.float32


def _prep_body(feat_ref, wemb_ref, bemb_ref, x0_ref, h_ref, lm_ref):
    h_ref[...] = (
        jnp.dot(feat_ref[...], wemb_ref[...], preferred_element_type=_f32)
        + bemb_ref[...]
    )
    lm_ref[...] = jnp.mean(x0_ref[...], axis=0, keepdims=True)


def _bias_body(glr_ref, glc_ref, ea_ref, wea_ref, wld_ref, be1_ref, out_ref):
    d = glr_ref[:, 0:3] - glc_ref[:, 0:3]
    ld = jnp.sum(d * d, axis=1, keepdims=True)
    eav = ea_ref[...]
    for l in range(2):
        out_ref[l] = (
            jnp.dot(eav, wea_ref[l], preferred_element_type=_f32)
            + ld * wld_ref[l]
            + be1_ref[l]
        )


_K1_BLK = 400


def _k1_body(h_ref, x_ref, v_ref, lm_ref, mh_ref, mx_ref, wa_ref, wb_ref,
             wv_ref, bv_ref, hn_ref, gr_ref, gc_ref, gx_ref, xn_ref, vn_ref,
             sv_ref):
    hin = h_ref[...]
    mhv = mh_ref[...]
    mxv = mx_ref[...]
    xc = x_ref[...] - lm_ref[...]
    vv = v_ref[...]
    pad = jnp.zeros((_K1_BLK, _XW - 3), _f32)
    for to in range(_T):
        acc = jnp.zeros((_K1_BLK, _D), _f32)
        for ti in range(_T):
            acc = acc + jnp.dot(hin[ti], mhv[ti, to],
                                preferred_element_type=_f32)
        hn_ref[to] = acc
        ax = jnp.zeros((_K1_BLK, 3), _f32)
        av = jnp.zeros((_K1_BLK, 3), _f32)
        for ti in range(_T):
            ax = ax + xc[ti] * mxv[ti, to, 0, 0] + vv[ti] * mxv[ti, to, 1, 0]
            av = av + xc[ti] * mxv[ti, to, 0, 1] + vv[ti] * mxv[ti, to, 1, 1]
        xo = ax + lm_ref[to]
        xn_ref[to] = xo
        vn_ref[to] = av
        gr_ref[to] = jnp.dot(acc, wa_ref[...],
                             preferred_element_type=_f32).astype(jnp.bfloat16)
        gc_ref[to] = jnp.dot(acc, wb_ref[...],
                             preferred_element_type=_f32).astype(jnp.bfloat16)
        gx_ref[to] = jnp.concatenate([xo, pad], axis=1)
        sv_ref[to] = (jnp.dot(acc, wv_ref[...], preferred_element_type=_f32)
                      + bv_ref[...])


_EB = 2000


def _make_edge_body(last):
    def body(gr_ref, gc_ref, xr_ref, xc_ref, b_ref, wr_ref, w2_ref, b2_ref,
             x1_ref, bx1_ref, x2_ref, out_ref):
        a = gr_ref[...].astype(_f32) + gc_ref[...].astype(_f32)
        d = xr_ref[:, 0:3] - xc_ref[:, 0:3]
        radial = jnp.sum(d * d, axis=1, keepdims=True)
        pre = a + radial * wr_ref[...] + b_ref[...]
        pre = pre * jax.nn.sigmoid(pre)
        m = jnp.dot(pre, w2_ref[...], preferred_element_type=_f32) + b2_ref[...]
        m = m * jax.nn.sigmoid(m)
        if last:
            t1 = (jnp.dot(m, x1_ref[...], preferred_element_type=_f32)
                  + bx1_ref[...])
            t1 = t1 * jax.nn.sigmoid(t1)
            tx = jnp.dot(t1, x2_ref[...], preferred_element_type=_f32)
            out_ref[...] = jnp.concatenate(
                [d * tx, jnp.ones((_EB, 1), _f32), jnp.zeros((_EB, 12), _f32)],
                axis=1)
        else:
            out_ref[...] = m
    return body


def _hupd_body(p_ref, hn_ref, wa_ref, wb_ref, b1_ref, w2_ref, b2_ref, out_ref):
    magg = p_ref[0] + p_ref[1]
    hv = hn_ref[...]
    u = (jnp.dot(hv, wa_ref[...], preferred_element_type=_f32)
         + jnp.dot(magg, wb_ref[...], preferred_element_type=_f32)
         + b1_ref[...])
    u = u * jax.nn.sigmoid(u)
    out_ref[...] = (hv + jnp.dot(u, w2_ref[...], preferred_element_type=_f32)
                    + b2_ref[...])


def _xupd_body(p_ref, xn_ref, vn_ref, sv_ref, out_ref):
    acc = p_ref[0] + p_ref[1]
    cnt = jnp.maximum(acc[:, 3:4], 1.0)
    agg = acc[:, 0:3] / cnt
    vout = sv_ref[...] * vn_ref[...] + agg
    out_ref[...] = xn_ref[...] + vout


def _make_gather_edge():
    total = _ET // _CH
    cpt = _BE // _CH
    niter = total // _NW
    out_h = jax.ShapeDtypeStruct((_ET, _HW), jnp.bfloat16)
    out_x = jax.ShapeDtypeStruct((_ET, _XW), _f32)
    mesh = plsc.VectorSubcoreMesh(core_axis_name="c", subcore_axis_name="s")

    @functools.partial(
        pl.kernel, mesh=mesh,
        out_type=(out_h, out_h, out_x, out_x),
        compiler_params=pltpu.CompilerParams(use_tc_tiling_on_sc=False),
        scratch_types=[
            pltpu.VMEM((_CH,), jnp.int32),
            pltpu.VMEM((_CH, _HW), jnp.bfloat16),
            pltpu.VMEM((_CH, _XW), _f32),
            pltpu.SemaphoreType.DMA,
        ],
    )
    def k(tab_hr, tab_hc, tab_x, idx_r, idx_c, o_hr, o_hc, o_xr, o_xc,
          idx_v, hrow_v, xrow_v, sem):
        wid = lax.axis_index("s") * 2 + lax.axis_index("c")

        def run_side(tab_h, idx, o_h, o_x):
            def body(kk, carry):
                g = wid + _NW * kk
                t = g // cpt
                j = g - t * cpt
                pltpu.sync_copy(idx.at[pl.ds(j * _CH, _CH)], idx_v)
                for i in range(_CH // 16):
                    idx_v[pl.ds(i * 16, 16)] = idx_v[pl.ds(i * 16, 16)] + t * _N
                pltpu.async_copy(tab_h.at[idx_v], hrow_v, sem).wait()
                pltpu.sync_copy(hrow_v, o_h.at[pl.ds(g * _CH, _CH)])
                pltpu.async_copy(tab_x.at[idx_v], xrow_v, sem).wait()
                pltpu.sync_copy(xrow_v, o_x.at[pl.ds(g * _CH, _CH)])
                return carry

            lax.fori_loop(0, niter, body, 0)

        run_side(tab_hr, idx_r, o_hr, o_xr)
        run_side(tab_hc, idx_c, o_hc, o_xc)

    return k


def _make_gather(n_tab, width, n_idx, t_rep, n_off):
    total = (n_idx * t_rep) // _CH
    cpt = n_idx // _CH
    niter = -(-total // _NW)
    out_sh = jax.ShapeDtypeStruct((n_idx * t_rep, width), _f32)
    mesh = plsc.VectorSubcoreMesh(core_axis_name="c", subcore_axis_name="s")

    @functools.partial(
        pl.kernel, mesh=mesh,
        out_type=(out_sh, out_sh),
        compiler_params=pltpu.CompilerParams(use_tc_tiling_on_sc=False),
        scratch_types=[
            pltpu.VMEM((_CH,), jnp.int32),
            pltpu.VMEM((_CH, width), _f32),
            pltpu.SemaphoreType.DMA,
        ],
    )
    def k(tab_r, tab_c, idx_r, idx_c, out_r, out_c, idx_v, row_v, sem):
        wid = lax.axis_index("s") * 2 + lax.axis_index("c")

        def run_side(tab, idx, out):
            def body(kk, carry):
                g = wid + _NW * kk

                @pl.when(g < total)
                def _():
                    t = g // cpt
                    j = g - t * cpt
                    pltpu.sync_copy(idx.at[pl.ds(j * _CH, _CH)], idx_v)
                    if t_rep > 1:
                        off = t * n_off
                        for i in range(_CH // 16):
                            idx_v[pl.ds(i * 16, 16)] = (
                                idx_v[pl.ds(i * 16, 16)] + off)
                    pltpu.async_copy(tab.at[idx_v], row_v, sem).wait()
                    pltpu.sync_copy(row_v, out.at[pl.ds(g * _CH, _CH)])
                return carry

            lax.fori_loop(0, niter, body, 0)

        run_side(tab_r, idx_r, out_r)
        run_side(tab_c, idx_c, out_c)

    return k


_SEG = 9216
_SROWS = _SEG // 16


def _make_scatter(width):
    total = _ET // _CH
    cpt = _BE // _CH
    niter = -(-total // _NW)
    rows = _SROWS
    mesh = plsc.VectorSubcoreMesh(core_axis_name="c", subcore_axis_name="s")

    @functools.partial(
        pl.kernel, mesh=mesh,
        out_type=jax.ShapeDtypeStruct((2 * _SEG, width), _f32),
        compiler_params=pltpu.CompilerParams(use_tc_tiling_on_sc=False),
        scratch_types=[
            pltpu.VMEM((_CH,), jnp.int32),
            pltpu.VMEM((_CH, width), _f32),
            pltpu.VMEM((rows, width), _f32),
            pltpu.VMEM_SHARED((_SEG, width), _f32),
        ],
    )
    def k(val_hbm, idx_hbm, zero_hbm, out_hbm, idx_v, val_v, blk_v, shared):
        cid = lax.axis_index("c")
        sid = lax.axis_index("s")
        wid = sid * 2 + cid
        pltpu.sync_copy(zero_hbm, shared.at[pl.ds(sid * rows, rows)])
        plsc.subcore_barrier()

        def body(kk, carry):
            g = wid + _NW * kk

            @pl.when(g < total)
            def _():
                t = g // cpt
                j = g - t * cpt
                pltpu.sync_copy(idx_hbm.at[pl.ds(j * _CH, _CH)], idx_v)
                off = t * _N
                for i in range(_CH // 16):
                    idx_v[pl.ds(i * 16, 16)] = idx_v[pl.ds(i * 16, 16)] + off
                pltpu.sync_copy(val_hbm.at[pl.ds(g * _CH, _CH)], val_v)
                pltpu.sync_copy(val_v, shared.at[idx_v], add=True)
            return carry

        lax.fori_loop(0, niter, body, 0)
        plsc.subcore_barrier()
        pltpu.sync_copy(shared.at[pl.ds(sid * rows, rows)], blk_v)
        pltpu.sync_copy(blk_v, out_hbm.at[pl.ds(cid * _SEG + sid * rows, rows)])

    return k


_make_gather = functools.lru_cache(maxsize=None)(_make_gather)
_make_gather_edge = functools.lru_cache(maxsize=None)(_make_gather_edge)
_make_scatter = functools.lru_cache(maxsize=None)(_make_scatter)


def _gather_loc(*args):
    return _make_gather(_BN, 16, _BE, 1, 0)(*args)


def _gather_edge(*args):
    return _make_gather_edge()(*args)


def _scatter_m(*args):
    return _make_scatter(_D)(*args)


def _scatter_x(*args):
    return _make_scatter(16)(*args)


def _timestep_embedding(num_timesteps, dim):
    half = dim // 2
    emb = math.log(10000.0) / (half - 1)
    freqs = jnp.exp(jnp.arange(half, dtype=_f32) * -emb)
    te = jnp.arange(num_timesteps, dtype=_f32)[:, None] * freqs[None, :]
    return jnp.concatenate([jnp.sin(te), jnp.cos(te)], axis=-1)


def _fold_time_conv(wr, wi):
    k = jnp.arange(_MODES, dtype=_f32)
    t = jnp.arange(_T, dtype=_f32)
    w_k = jnp.where(k == 0, 1.0, 2.0)
    ang = (2.0 * math.pi / _T) * k[None, None, :] * (
        t[None, :, None] - t[:, None, None])
    m = (jnp.einsum('itk,k,kco->itco', jnp.cos(ang), w_k, wr)
         - jnp.einsum('itk,k,kco->itco', jnp.sin(ang), w_k, wi)) / _T
    eye = jnp.eye(wr.shape[1], dtype=_f32)
    return m + jnp.eye(_T, dtype=_f32)[:, :, None, None] * eye[None, None]


def kernel(x_0, v_0, concatenated_features, edge_attr, source_node_indices,
           target_node_indices, params):
    P = params['layers']

    te = _timestep_embedding(_T, _TIME_EMB)
    time_emb = jnp.reshape(
        jnp.broadcast_to(te[:, None, :], (_T, _BN, _TIME_EMB)),
        (_BTN, _TIME_EMB))
    h2 = jnp.reshape(concatenated_features[..., -2:], (_BTN, 2))
    feat = jnp.concatenate([h2, time_emb], axis=1)
    x = jnp.reshape(x_0[..., :3], (_BTN, 3))
    v = jnp.reshape(v_0[..., :3], (_BTN, 3))
    loc = jnp.reshape(x_0[:, 0, :, :3], (_BN, 3))
    locpad = jnp.concatenate([loc, jnp.zeros((_BN, 13), _f32)], axis=1)
    rows0 = jnp.reshape(source_node_indices, (_BE,)).astype(jnp.int32)
    cols0 = jnp.reshape(target_node_indices, (_BE,)).astype(jnp.int32)
    ea = jnp.reshape(edge_attr, (_BE, 2))

    mh = [_fold_time_conv(P[l]['wt_r'], P[l]['wt_i']) for l in range(2)]
    mx = [_fold_time_conv(P[l]['wtx_r'], P[l]['wtx_i']) for l in range(2)]
    wea_s = jnp.stack([P[l]['We1'][129:131] for l in range(2)])
    wld_s = jnp.stack([P[l]['We1'][131] for l in range(2)])
    be1_s = jnp.stack([P[l]['be1'] for l in range(2)])
    z64 = jnp.zeros((_SROWS, _D), _f32)
    z16 = jnp.zeros((_SROWS, 16), _f32)

    x0p = jnp.reshape(jnp.transpose(x_0[..., :3], (2, 0, 1, 3)), (_N, 48))
    h, lm48 = pl.pallas_call(
        _prep_body,
        out_shape=(jax.ShapeDtypeStruct((_BTN, _D), _f32),
                   jax.ShapeDtypeStruct((1, 48), _f32)),
    )(feat, params['W_emb'], params['b_emb'][None, :], x0p)
    lm = jnp.reshape(
        jnp.broadcast_to(jnp.reshape(lm48, (_B * _T, 1, 3)), (_B * _T, _N, 3)),
        (_BTN, 3))

    gl_r, gl_c = _gather_loc(locpad, locpad, rows0, cols0)

    bias = pl.pallas_call(
        _bias_body,
        grid=(8,),
        in_specs=[
            pl.BlockSpec((_BE // 8, 16), lambda i: (i, 0)),
            pl.BlockSpec((_BE // 8, 16), lambda i: (i, 0)),
            pl.BlockSpec((_BE // 8, 2), lambda i: (i, 0)),
            pl.BlockSpec((2, 2, _D), lambda i: (0, 0, 0)),
            pl.BlockSpec((2, _D), lambda i: (0, 0)),
            pl.BlockSpec((2, _D), lambda i: (0, 0)),
        ],
        out_specs=pl.BlockSpec((2, _BE // 8, _D), lambda i: (0, i, 0)),
        out_shape=jax.ShapeDtypeStruct((2, _BE, _D), _f32),
    )(gl_r, gl_c, ea, wea_s, wld_s, be1_s)

    x_out = None
    for l in range(2):
        Pl = P[l]
        last = l == 1
        k1_out = pl.pallas_call(
            _k1_body,
            grid=(_BN // _K1_BLK,),
            in_specs=[
                pl.BlockSpec((_T, _K1_BLK, _D), lambda i: (0, i, 0)),
                pl.BlockSpec((_T, _K1_BLK, 3), lambda i: (0, i, 0)),
                pl.BlockSpec((_T, _K1_BLK, 3), lambda i: (0, i, 0)),
                pl.BlockSpec((_T, _K1_BLK, 3), lambda i: (0, i, 0)),
                pl.BlockSpec((_T, _T, _D, _D), lambda i: (0, 0, 0, 0)),
                pl.BlockSpec((_T, _T, 2, 2), lambda i: (0, 0, 0, 0)),
                pl.BlockSpec((_D, _D), lambda i: (0, 0)),
                pl.BlockSpec((_D, _D), lambda i: (0, 0)),
                pl.BlockSpec((_D, 1), lambda i: (0, 0)),
                pl.BlockSpec((1, 1), lambda i: (0, 0)),
            ],
            out_specs=[
                pl.BlockSpec((_T, _K1_BLK, _D), lambda i: (0, i, 0)),
                pl.BlockSpec((_T, _K1_BLK, _HW), lambda i: (0, i, 0)),
                pl.BlockSpec((_T, _K1_BLK, _HW), lambda i: (0, i, 0)),
                pl.BlockSpec((_T, _K1_BLK, _XW), lambda i: (0, i, 0)),
                pl.BlockSpec((_T, _K1_BLK, 3), lambda i: (0, i, 0)),
                pl.BlockSpec((_T, _K1_BLK, 3), lambda i: (0, i, 0)),
                pl.BlockSpec((_T, _K1_BLK, 1), lambda i: (0, i, 0)),
            ],
            out_shape=(
                jax.ShapeDtypeStruct((_T, _BN, _D), _f32),
                jax.ShapeDtypeStruct((_T, _BN, _HW), jnp.bfloat16),
                jax.ShapeDtypeStruct((_T, _BN, _HW), jnp.bfloat16),
                jax.ShapeDtypeStruct((_T, _BN, _XW), _f32),
                jax.ShapeDtypeStruct((_T, _BN, 3), _f32),
                jax.ShapeDtypeStruct((_T, _BN, 3), _f32),
                jax.ShapeDtypeStruct((_T, _BN, 1), _f32),
            ),
        )(jnp.reshape(h, (_T, _BN, _D)), jnp.reshape(x, (_T, _BN, 3)),
          jnp.reshape(v, (_T, _BN, 3)), jnp.reshape(lm, (_T, _BN, 3)),
          mh[l], mx[l], Pl['We1'][0:_D], Pl['We1'][_D:2 * _D], Pl['Wv'],
          Pl['bv'][None, :])
        hn_t, gr_t, gc_t, gx_t, xn_t, vn_t, sv_t = k1_out
        hn = jnp.reshape(hn_t, (_BTN, _D))
        tab_hr = jnp.reshape(gr_t, (_BTN, _HW))
        tab_hc = jnp.reshape(gc_t, (_BTN, _HW))
        tab_x = jnp.reshape(gx_t, (_BTN, _XW))
        xn = jnp.reshape(xn_t, (_BTN, 3))
        vn = jnp.reshape(vn_t, (_BTN, 3))

        g_hr, g_hc, g_xr, g_xc = _gather_edge(tab_hr, tab_hc, tab_x,
                                              rows0, cols0)

        w_out = 16 if last else _D
        val = pl.pallas_call(
            _make_edge_body(last),
            grid=(_ET // _EB,),
            in_specs=[
                pl.BlockSpec((_EB, _HW), lambda i: (i, 0)),
                pl.BlockSpec((_EB, _HW), lambda i: (i, 0)),
                pl.BlockSpec((_EB, _XW), lambda i: (i, 0)),
                pl.BlockSpec((_EB, _XW), lambda i: (i, 0)),
                pl.BlockSpec((_EB, _D), lambda i: (i % (_BE // _EB), 0)),
                pl.BlockSpec((1, _D), lambda i: (0, 0)),
                pl.BlockSpec((_D, _D), lambda i: (0, 0)),
                pl.BlockSpec((1, _D), lambda i: (0, 0)),
                pl.BlockSpec((_D, _D), lambda i: (0, 0)),
                pl.BlockSpec((1, _D), lambda i: (0, 0)),
                pl.BlockSpec((_D, 1), lambda i: (0, 0)),
            ],
            out_specs=pl.BlockSpec((_EB, w_out), lambda i: (i, 0)),
            out_shape=jax.ShapeDtypeStruct((_ET, w_out), _f32),
        )(g_hr, g_hc, g_xr, g_xc, bias[l], Pl['We1'][128][None, :], Pl['We2'],
          Pl['be2'][None, :], Pl['Wx1'], Pl['bx1'][None, :], Pl['Wx2'])

        if not last:
            part = _scatter_m(val, rows0, z64)
            p = jnp.concatenate(
                [jnp.reshape(part, (2, _SEG, _D)),
                 jnp.zeros((2, _BTN - _SEG, _D), _f32)], axis=1)
            h = pl.pallas_call(
                _hupd_body,
                grid=(8,),
                in_specs=[
                    pl.BlockSpec((2, _BTN // 8, _D), lambda i: (0, i, 0)),
                    pl.BlockSpec((_BTN // 8, _D), lambda i: (i, 0)),
                    pl.BlockSpec((_D, _D), lambda i: (0, 0)),
                    pl.BlockSpec((_D, _D), lambda i: (0, 0)),
                    pl.BlockSpec((1, _D), lambda i: (0, 0)),
                    pl.BlockSpec((_D, _D), lambda i: (0, 0)),
                    pl.BlockSpec((1, _D), lambda i: (0, 0)),
                ],
                out_specs=pl.BlockSpec((_BTN // 8, _D), lambda i: (i, 0)),
                out_shape=jax.ShapeDtypeStruct((_BTN, _D), _f32),
            )(p, hn, Pl['Wh1'][0:_D], Pl['Wh1'][_D:2 * _D],
              Pl['bh1'][None, :], Pl['Wh2'], Pl['bh2'][None, :])
            x, v = xn, vn
        else:
            part = _scatter_x(val, rows0, z16)
            p = jnp.concatenate(
                [jnp.reshape(part, (2, _SEG, 16)),
                 jnp.zeros((2, _BTN - _SEG, 16), _f32)], axis=1)
            x_out = pl.pallas_call(
                _xupd_body,
                grid=(8,),
                in_specs=[
                    pl.BlockSpec((2, _BTN // 8, 16), lambda i: (0, i, 0)),
                    pl.BlockSpec((_BTN // 8, 3), lambda i: (i, 0)),
                    pl.BlockSpec((_BTN // 8, 3), lambda i: (i, 0)),
                    pl.BlockSpec((_BTN // 8, 1), lambda i: (i, 0)),
                ],
                out_specs=pl.BlockSpec((_BTN // 8, 3), lambda i: (i, 0)),
                out_shape=jax.ShapeDtypeStruct((_BTN, 3), _f32),
            )(p, xn, vn, jnp.reshape(sv_t, (_BTN, 1)))

    return jnp.reshape(x_out, (_B, _T, _N, 3))


# speedup vs baseline: 1.0552x; 1.0552x over previous
import functools
import math

import jax
import jax.numpy as jnp
from jax import lax
from jax.experimental import pallas as pl
from jax.experimental.pallas import tpu as pltpu
from jax.experimental.pallas import tpu_sc as plsc

_B, _T, _N, _E = 2, 8, 1000, 16000
_D = 64
_MODES = 4
_TIME_EMB = 32
_BN = _B * _N
_BTN = _B * _T * _N
_BE = _B * _E
_ET = _T * _BE
_GW = 96
_XW = 16
_CH = 800
_NW = 32

_f32 = jnp.float32


def _prep_body(feat_ref, wemb_ref, bemb_ref, x0_ref, h_ref, lm_ref):
    h_ref[...] = (
        jnp.dot(feat_ref[...], wemb_ref[...], preferred_element_type=_f32)
        + bemb_ref[...]
    )
    lm_ref[...] = jnp.mean(x0_ref[...], axis=0, keepdims=True)


def _bias_body(glr_ref, glc_ref, ea_ref, wea_ref, wld_ref, be1_ref, out_ref):
    d = glr_ref[:, 0:3] - glc_ref[:, 0:3]
    ld = jnp.sum(d * d, axis=1, keepdims=True)
    eav = ea_ref[...]
    for l in range(2):
        out_ref[l] = (
            jnp.dot(eav, wea_ref[l], preferred_element_type=_f32)
            + ld * wld_ref[l]
            + be1_ref[l]
        )


_K1_BLK = 400


def _k1_body(h_ref, x_ref, v_ref, lm_ref, mh_ref, mx_ref, wa_ref, wb_ref,
             wv_ref, bv_ref, hn_ref, gr_ref, gc_ref, xn_ref, vn_ref,
             sv_ref):
    hin = h_ref[...]
    mhv = mh_ref[...]
    mxv = mx_ref[...]
    xc = x_ref[...] - lm_ref[...]
    vv = v_ref[...]
    pad = jnp.zeros((_K1_BLK, _GW - _D - 3), _f32)
    for to in range(_T):
        acc = jnp.zeros((_K1_BLK, _D), _f32)
        for ti in range(_T):
            acc = acc + jnp.dot(hin[ti], mhv[ti, to],
                                preferred_element_type=_f32)
        hn_ref[to] = acc
        ax = jnp.zeros((_K1_BLK, 3), _f32)
        av = jnp.zeros((_K1_BLK, 3), _f32)
        for ti in range(_T):
            ax = ax + xc[ti] * mxv[ti, to, 0, 0] + vv[ti] * mxv[ti, to, 1, 0]
            av = av + xc[ti] * mxv[ti, to, 0, 1] + vv[ti] * mxv[ti, to, 1, 1]
        xo = ax + lm_ref[to]
        xn_ref[to] = xo
        vn_ref[to] = av
        gr_ref[to] = jnp.concatenate(
            [jnp.dot(acc, wa_ref[...], preferred_element_type=_f32), xo, pad],
            axis=1).astype(jnp.bfloat16)
        gc_ref[to] = jnp.concatenate(
            [jnp.dot(acc, wb_ref[...], preferred_element_type=_f32), xo, pad],
            axis=1).astype(jnp.bfloat16)
        sv_ref[to] = (jnp.dot(acc, wv_ref[...], preferred_element_type=_f32)
                      + bv_ref[...])


_EB = 2000


def _make_edge_body(last):
    def body(gr_ref, gc_ref, b_ref, wr_ref, w2_ref, b2_ref,
             x1_ref, bx1_ref, x2_ref, out_ref):
        grv = gr_ref[...].astype(_f32)
        gcv = gc_ref[...].astype(_f32)
        a = grv[:, 0:_D] + gcv[:, 0:_D]
        d = grv[:, _D:_D + 3] - gcv[:, _D:_D + 3]
        radial = jnp.sum(d * d, axis=1, keepdims=True)
        pre = a + radial * wr_ref[...] + b_ref[...]
        pre = pre * jax.nn.sigmoid(pre)
        m = jnp.dot(pre, w2_ref[...], preferred_element_type=_f32) + b2_ref[...]
        m = m * jax.nn.sigmoid(m)
        if last:
            t1 = (jnp.dot(m, x1_ref[...], preferred_element_type=_f32)
                  + bx1_ref[...])
            t1 = t1 * jax.nn.sigmoid(t1)
            tx = jnp.dot(t1, x2_ref[...], preferred_element_type=_f32)
            out_ref[...] = jnp.concatenate(
                [d * tx, jnp.ones((_EB, 1), _f32), jnp.zeros((_EB, 12), _f32)],
                axis=1)
        else:
            out_ref[...] = m
    return body


def _hupd_body(p_ref, hn_ref, wa_ref, wb_ref, b1_ref, w2_ref, b2_ref, out_ref):
    magg = p_ref[0] + p_ref[1]
    hv = hn_ref[...]
    u = (jnp.dot(hv, wa_ref[...], preferred_element_type=_f32)
         + jnp.dot(magg, wb_ref[...], preferred_element_type=_f32)
         + b1_ref[...])
    u = u * jax.nn.sigmoid(u)
    out_ref[...] = (hv + jnp.dot(u, w2_ref[...], preferred_element_type=_f32)
                    + b2_ref[...])


def _xupd_body(p_ref, xn_ref, vn_ref, sv_ref, out_ref):
    acc = p_ref[0] + p_ref[1]
    cnt = jnp.maximum(acc[:, 3:4], 1.0)
    agg = acc[:, 0:3] / cnt
    vout = sv_ref[...] * vn_ref[...] + agg
    out_ref[...] = xn_ref[...] + vout


def _make_gather_edge():
    total = _ET // _CH
    cpt = _BE // _CH
    niter = total // _NW
    out_h = jax.ShapeDtypeStruct((_ET, _GW), jnp.bfloat16)
    mesh = plsc.VectorSubcoreMesh(core_axis_name="c", subcore_axis_name="s")

    @functools.partial(
        pl.kernel, mesh=mesh,
        out_type=(out_h, out_h),
        compiler_params=pltpu.CompilerParams(use_tc_tiling_on_sc=False),
        scratch_types=[
            pltpu.VMEM((_CH,), jnp.int32),
            pltpu.VMEM((_CH, _GW), jnp.bfloat16),
            pltpu.SemaphoreType.DMA,
        ],
    )
    def k(tab_hr, tab_hc, idx_r, idx_c, o_hr, o_hc, idx_v, hrow_v, sem):
        wid = lax.axis_index("s") * 2 + lax.axis_index("c")

        def run_side(tab_h, idx, o_h):
            def body(kk, carry):
                g = wid + _NW * kk
                t = g // cpt
                j = g - t * cpt
                pltpu.sync_copy(idx.at[pl.ds(j * _CH, _CH)], idx_v)
                for i in range(_CH // 16):
                    idx_v[pl.ds(i * 16, 16)] = idx_v[pl.ds(i * 16, 16)] + t * _N
                pltpu.async_copy(tab_h.at[idx_v], hrow_v, sem).wait()
                pltpu.sync_copy(hrow_v, o_h.at[pl.ds(g * _CH, _CH)])
                return carry

            lax.fori_loop(0, niter, body, 0)

        run_side(tab_hr, idx_r, o_hr)
        run_side(tab_hc, idx_c, o_hc)

    return k


def _make_gather(n_tab, width, n_idx, t_rep, n_off):
    total = (n_idx * t_rep) // _CH
    cpt = n_idx // _CH
    niter = -(-total // _NW)
    out_sh = jax.ShapeDtypeStruct((n_idx * t_rep, width), _f32)
    mesh = plsc.VectorSubcoreMesh(core_axis_name="c", subcore_axis_name="s")

    @functools.partial(
        pl.kernel, mesh=mesh,
        out_type=(out_sh, out_sh),
        compiler_params=pltpu.CompilerParams(use_tc_tiling_on_sc=False),
        scratch_types=[
            pltpu.VMEM((_CH,), jnp.int32),
            pltpu.VMEM((_CH, width), _f32),
            pltpu.SemaphoreType.DMA,
        ],
    )
    def k(tab_r, tab_c, idx_r, idx_c, out_r, out_c, idx_v, row_v, sem):
        wid = lax.axis_index("s") * 2 + lax.axis_index("c")

        def run_side(tab, idx, out):
            def body(kk, carry):
                g = wid + _NW * kk

                @pl.when(g < total)
                def _():
                    t = g // cpt
                    j = g - t * cpt
                    pltpu.sync_copy(idx.at[pl.ds(j * _CH, _CH)], idx_v)
                    if t_rep > 1:
                        off = t * n_off
                        for i in range(_CH // 16):
                            idx_v[pl.ds(i * 16, 16)] = (
                                idx_v[pl.ds(i * 16, 16)] + off)
                    pltpu.async_copy(tab.at[idx_v], row_v, sem).wait()
                    pltpu.sync_copy(row_v, out.at[pl.ds(g * _CH, _CH)])
                return carry

            lax.fori_loop(0, niter, body, 0)

        run_side(tab_r, idx_r, out_r)
        run_side(tab_c, idx_c, out_c)

    return k


_SEG = 9216
_SROWS = _SEG // 16


def _make_scatter(width):
    total = _ET // _CH
    cpt = _BE // _CH
    niter = -(-total // _NW)
    rows = _SROWS
    mesh = plsc.VectorSubcoreMesh(core_axis_name="c", subcore_axis_name="s")

    @functools.partial(
        pl.kernel, mesh=mesh,
        out_type=jax.ShapeDtypeStruct((2 * _SEG, width), _f32),
        compiler_params=pltpu.CompilerParams(use_tc_tiling_on_sc=False),
        scratch_types=[
            pltpu.VMEM((_CH,), jnp.int32),
            pltpu.VMEM((_CH, width), _f32),
            pltpu.VMEM((rows, width), _f32),
            pltpu.VMEM_SHARED((_SEG, width), _f32),
        ],
    )
    def k(val_hbm, idx_hbm, zero_hbm, out_hbm, idx_v, val_v, blk_v, shared):
        cid = lax.axis_index("c")
        sid = lax.axis_index("s")
        wid = sid * 2 + cid
        pltpu.sync_copy(zero_hbm, shared.at[pl.ds(sid * rows, rows)])
        plsc.subcore_barrier()

        def body(kk, carry):
            g = wid + _NW * kk

            @pl.when(g < total)
            def _():
                t = g // cpt
                j = g - t * cpt
                pltpu.sync_copy(idx_hbm.at[pl.ds(j * _CH, _CH)], idx_v)
                off = t * _N
                for i in range(_CH // 16):
                    idx_v[pl.ds(i * 16, 16)] = idx_v[pl.ds(i * 16, 16)] + off
                pltpu.sync_copy(val_hbm.at[pl.ds(g * _CH, _CH)], val_v)
                pltpu.sync_copy(val_v, shared.at[idx_v], add=True)
            return carry

        lax.fori_loop(0, niter, body, 0)
        plsc.subcore_barrier()
        pltpu.sync_copy(shared.at[pl.ds(sid * rows, rows)], blk_v)
        pltpu.sync_copy(blk_v, out_hbm.at[pl.ds(cid * _SEG + sid * rows, rows)])

    return k


_make_gather = functools.lru_cache(maxsize=None)(_make_gather)
_make_gather_edge = functools.lru_cache(maxsize=None)(_make_gather_edge)
_make_scatter = functools.lru_cache(maxsize=None)(_make_scatter)


def _gather_loc(*args):
    return _make_gather(_BN, 16, _BE, 1, 0)(*args)


def _gather_edge(*args):
    return _make_gather_edge()(*args)


def _scatter_m(*args):
    return _make_scatter(_D)(*args)


def _scatter_x(*args):
    return _make_scatter(16)(*args)


def _timestep_embedding(num_timesteps, dim):
    half = dim // 2
    emb = math.log(10000.0) / (half - 1)
    freqs = jnp.exp(jnp.arange(half, dtype=_f32) * -emb)
    te = jnp.arange(num_timesteps, dtype=_f32)[:, None] * freqs[None, :]
    return jnp.concatenate([jnp.sin(te), jnp.cos(te)], axis=-1)


def _fold_time_conv(wr, wi):
    k = jnp.arange(_MODES, dtype=_f32)
    t = jnp.arange(_T, dtype=_f32)
    w_k = jnp.where(k == 0, 1.0, 2.0)
    ang = (2.0 * math.pi / _T) * k[None, None, :] * (
        t[None, :, None] - t[:, None, None])
    m = (jnp.einsum('itk,k,kco->itco', jnp.cos(ang), w_k, wr)
         - jnp.einsum('itk,k,kco->itco', jnp.sin(ang), w_k, wi)) / _T
    eye = jnp.eye(wr.shape[1], dtype=_f32)
    return m + jnp.eye(_T, dtype=_f32)[:, :, None, None] * eye[None, None]


def kernel(x_0, v_0, concatenated_features, edge_attr, source_node_indices,
           target_node_indices, params):
    P = params['layers']

    te = _timestep_embedding(_T, _TIME_EMB)
    time_emb = jnp.reshape(
        jnp.broadcast_to(te[:, None, :], (_T, _BN, _TIME_EMB)),
        (_BTN, _TIME_EMB))
    h2 = jnp.reshape(concatenated_features[..., -2:], (_BTN, 2))
    feat = jnp.concatenate([h2, time_emb], axis=1)
    x = jnp.reshape(x_0[..., :3], (_BTN, 3))
    v = jnp.reshape(v_0[..., :3], (_BTN, 3))
    loc = jnp.reshape(x_0[:, 0, :, :3], (_BN, 3))
    locpad = jnp.concatenate([loc, jnp.zeros((_BN, 13), _f32)], axis=1)
    rows0 = jnp.reshape(source_node_indices, (_BE,)).astype(jnp.int32)
    cols0 = jnp.reshape(target_node_indices, (_BE,)).astype(jnp.int32)
    ea = jnp.reshape(edge_attr, (_BE, 2))

    mh = [_fold_time_conv(P[l]['wt_r'], P[l]['wt_i']) for l in range(2)]
    mx = [_fold_time_conv(P[l]['wtx_r'], P[l]['wtx_i']) for l in range(2)]
    wea_s = jnp.stack([P[l]['We1'][129:131] for l in range(2)])
    wld_s = jnp.stack([P[l]['We1'][131] for l in range(2)])
    be1_s = jnp.stack([P[l]['be1'] for l in range(2)])
    z64 = jnp.zeros((_SROWS, _D), _f32)
    z16 = jnp.zeros((_SROWS, 16), _f32)

    x0p = jnp.reshape(jnp.transpose(x_0[..., :3], (2, 0, 1, 3)), (_N, 48))
    h, lm48 = pl.pallas_call(
        _prep_body,
        out_shape=(jax.ShapeDtypeStruct((_BTN, _D), _f32),
                   jax.ShapeDtypeStruct((1, 48), _f32)),
    )(feat, params['W_emb'], params['b_emb'][None, :], x0p)
    lm = jnp.reshape(
        jnp.broadcast_to(jnp.reshape(lm48, (_B * _T, 1, 3)), (_B * _T, _N, 3)),
        (_BTN, 3))

    gl_r, gl_c = _gather_loc(locpad, locpad, rows0, cols0)

    bias = pl.pallas_call(
        _bias_body,
        grid=(8,),
        in_specs=[
            pl.BlockSpec((_BE // 8, 16), lambda i: (i, 0)),
            pl.BlockSpec((_BE // 8, 16), lambda i: (i, 0)),
            pl.BlockSpec((_BE // 8, 2), lambda i: (i, 0)),
            pl.BlockSpec((2, 2, _D), lambda i: (0, 0, 0)),
            pl.BlockSpec((2, _D), lambda i: (0, 0)),
            pl.BlockSpec((2, _D), lambda i: (0, 0)),
        ],
        out_specs=pl.BlockSpec((2, _BE // 8, _D), lambda i: (0, i, 0)),
        out_shape=jax.ShapeDtypeStruct((2, _BE, _D), _f32),
    )(gl_r, gl_c, ea, wea_s, wld_s, be1_s)

    x_out = None
    for l in range(2):
        Pl = P[l]
        last = l == 1
        k1_out = pl.pallas_call(
            _k1_body,
            grid=(_BN // _K1_BLK,),
            in_specs=[
                pl.BlockSpec((_T, _K1_BLK, _D), lambda i: (0, i, 0)),
                pl.BlockSpec((_T, _K1_BLK, 3), lambda i: (0, i, 0)),
                pl.BlockSpec((_T, _K1_BLK, 3), lambda i: (0, i, 0)),
                pl.BlockSpec((_T, _K1_BLK, 3), lambda i: (0, i, 0)),
                pl.BlockSpec((_T, _T, _D, _D), lambda i: (0, 0, 0, 0)),
                pl.BlockSpec((_T, _T, 2, 2), lambda i: (0, 0, 0, 0)),
                pl.BlockSpec((_D, _D), lambda i: (0, 0)),
                pl.BlockSpec((_D, _D), lambda i: (0, 0)),
                pl.BlockSpec((_D, 1), lambda i: (0, 0)),
                pl.BlockSpec((1, 1), lambda i: (0, 0)),
            ],
            out_specs=[
                pl.BlockSpec((_T, _K1_BLK, _D), lambda i: (0, i, 0)),
                pl.BlockSpec((_T, _K1_BLK, _GW), lambda i: (0, i, 0)),
                pl.BlockSpec((_T, _K1_BLK, _GW), lambda i: (0, i, 0)),
                pl.BlockSpec((_T, _K1_BLK, 3), lambda i: (0, i, 0)),
                pl.BlockSpec((_T, _K1_BLK, 3), lambda i: (0, i, 0)),
                pl.BlockSpec((_T, _K1_BLK, 1), lambda i: (0, i, 0)),
            ],
            out_shape=(
                jax.ShapeDtypeStruct((_T, _BN, _D), _f32),
                jax.ShapeDtypeStruct((_T, _BN, _GW), jnp.bfloat16),
                jax.ShapeDtypeStruct((_T, _BN, _GW), jnp.bfloat16),
                jax.ShapeDtypeStruct((_T, _BN, 3), _f32),
                jax.ShapeDtypeStruct((_T, _BN, 3), _f32),
                jax.ShapeDtypeStruct((_T, _BN, 1), _f32),
            ),
        )(jnp.reshape(h, (_T, _BN, _D)), jnp.reshape(x, (_T, _BN, 3)),
          jnp.reshape(v, (_T, _BN, 3)), jnp.reshape(lm, (_T, _BN, 3)),
          mh[l], mx[l], Pl['We1'][0:_D], Pl['We1'][_D:2 * _D], Pl['Wv'],
          Pl['bv'][None, :])
        hn_t, gr_t, gc_t, xn_t, vn_t, sv_t = k1_out
        hn = jnp.reshape(hn_t, (_BTN, _D))
        tab_hr = jnp.reshape(gr_t, (_BTN, _GW))
        tab_hc = jnp.reshape(gc_t, (_BTN, _GW))
        xn = jnp.reshape(xn_t, (_BTN, 3))
        vn = jnp.reshape(vn_t, (_BTN, 3))

        g_hr, g_hc = _gather_edge(tab_hr, tab_hc, rows0, cols0)

        w_out = 16 if last else _D
        val = pl.pallas_call(
            _make_edge_body(last),
            grid=(_ET // _EB,),
            in_specs=[
                pl.BlockSpec((_EB, _GW), lambda i: (i, 0)),
                pl.BlockSpec((_EB, _GW), lambda i: (i, 0)),
                pl.BlockSpec((_EB, _D), lambda i: (i % (_BE // _EB), 0)),
                pl.BlockSpec((1, _D), lambda i: (0, 0)),
                pl.BlockSpec((_D, _D), lambda i: (0, 0)),
                pl.BlockSpec((1, _D), lambda i: (0, 0)),
                pl.BlockSpec((_D, _D), lambda i: (0, 0)),
                pl.BlockSpec((1, _D), lambda i: (0, 0)),
                pl.BlockSpec((_D, 1), lambda i: (0, 0)),
            ],
            out_specs=pl.BlockSpec((_EB, w_out), lambda i: (i, 0)),
            out_shape=jax.ShapeDtypeStruct((_ET, w_out), _f32),
        )(g_hr, g_hc, bias[l], Pl['We1'][128][None, :], Pl['We2'],
          Pl['be2'][None, :], Pl['Wx1'], Pl['bx1'][None, :], Pl['Wx2'])

        if not last:
            part = _scatter_m(val, rows0, z64)
            p = jnp.concatenate(
                [jnp.reshape(part, (2, _SEG, _D)),
                 jnp.zeros((2, _BTN - _SEG, _D), _f32)], axis=1)
            h = pl.pallas_call(
                _hupd_body,
                grid=(8,),
                in_specs=[
                    pl.BlockSpec((2, _BTN // 8, _D), lambda i: (0, i, 0)),
                    pl.BlockSpec((_BTN // 8, _D), lambda i: (i, 0)),
                    pl.BlockSpec((_D, _D), lambda i: (0, 0)),
                    pl.BlockSpec((_D, _D), lambda i: (0, 0)),
                    pl.BlockSpec((1, _D), lambda i: (0, 0)),
                    pl.BlockSpec((_D, _D), lambda i: (0, 0)),
                    pl.BlockSpec((1, _D), lambda i: (0, 0)),
                ],
                out_specs=pl.BlockSpec((_BTN // 8, _D), lambda i: (i, 0)),
                out_shape=jax.ShapeDtypeStruct((_BTN, _D), _f32),
            )(p, hn, Pl['Wh1'][0:_D], Pl['Wh1'][_D:2 * _D],
              Pl['bh1'][None, :], Pl['Wh2'], Pl['bh2'][None, :])
            x, v = xn, vn
        else:
            part = _scatter_x(val, rows0, z16)
            p = jnp.concatenate(
                [jnp.reshape(part, (2, _SEG, 16)),
                 jnp.zeros((2, _BTN - _SEG, 16), _f32)], axis=1)
            x_out = pl.pallas_call(
                _xupd_body,
                grid=(8,),
                in_specs=[
                    pl.BlockSpec((2, _BTN // 8, 16), lambda i: (0, i, 0)),
                    pl.BlockSpec((_BTN // 8, 3), lambda i: (i, 0)),
                    pl.BlockSpec((_BTN // 8, 3), lambda i: (i, 0)),
                    pl.BlockSpec((_BTN // 8, 1), lambda i: (i, 0)),
                ],
                out_specs=pl.BlockSpec((_BTN // 8, 3), lambda i: (i, 0)),
                out_shape=jax.ShapeDtypeStruct((_BTN, 3), _f32),
            )(p, xn, vn, jnp.reshape(sv_t, (_BTN, 1)))

    return jnp.reshape(x_out, (_B, _T, _N, 3))


# speedup vs baseline: 1.2492x; 1.1838x over previous
import functools
import math

import jax
import jax.numpy as jnp
from jax import lax
from jax.experimental import pallas as pl
from jax.experimental.pallas import tpu as pltpu
from jax.experimental.pallas import tpu_sc as plsc

_B, _T, _N, _E = 2, 8, 1000, 16000
_D = 64
_MODES = 4
_TIME_EMB = 32
_BN = _B * _N
_BTN = _B * _T * _N
_BE = _B * _E
_ET = _T * _BE
_GW = 80
_XW = 16
_CH = 800
_NW = 32

_f32 = jnp.float32


def _prep_body(feat_ref, wemb_ref, bemb_ref, x0_ref, h_ref, lm_ref):
    h_ref[...] = (
        jnp.dot(feat_ref[...], wemb_ref[...], preferred_element_type=_f32)
        + bemb_ref[...]
    )
    lm_ref[...] = jnp.mean(x0_ref[...], axis=0, keepdims=True)


def _bias_body(glr_ref, glc_ref, ea_ref, wea_ref, wld_ref, be1_ref, out_ref):
    d = glr_ref[:, 0:3] - glc_ref[:, 0:3]
    ld = jnp.sum(d * d, axis=1, keepdims=True)
    eav = ea_ref[...]
    for l in range(2):
        out_ref[l] = (
            jnp.dot(eav, wea_ref[l], preferred_element_type=_f32)
            + ld * wld_ref[l]
            + be1_ref[l]
        )


_K1_BLK = 400


def _k1_body(h_ref, x_ref, v_ref, lm_ref, mh_ref, mx_ref, wa_ref, wb_ref,
             wv_ref, bv_ref, hn_ref, gr_ref, gc_ref, xn_ref, vn_ref,
             sv_ref):
    hin = h_ref[...]
    mhv = mh_ref[...]
    mxv = mx_ref[...]
    xc = x_ref[...] - lm_ref[...]
    vv = v_ref[...]
    pad = jnp.zeros((_K1_BLK, _GW - _D - 3), _f32)
    for to in range(_T):
        acc = jnp.zeros((_K1_BLK, _D), _f32)
        for ti in range(_T):
            acc = acc + jnp.dot(hin[ti], mhv[ti, to],
                                preferred_element_type=_f32)
        hn_ref[to] = acc
        ax = jnp.zeros((_K1_BLK, 3), _f32)
        av = jnp.zeros((_K1_BLK, 3), _f32)
        for ti in range(_T):
            ax = ax + xc[ti] * mxv[ti, to, 0, 0] + vv[ti] * mxv[ti, to, 1, 0]
            av = av + xc[ti] * mxv[ti, to, 0, 1] + vv[ti] * mxv[ti, to, 1, 1]
        xo = ax + lm_ref[to]
        xn_ref[to] = xo
        vn_ref[to] = av
        gr_ref[to] = jnp.concatenate(
            [jnp.dot(acc, wa_ref[...], preferred_element_type=_f32), xo, pad],
            axis=1)
        gc_ref[to] = jnp.concatenate(
            [jnp.dot(acc, wb_ref[...], preferred_element_type=_f32), xo, pad],
            axis=1)
        sv_ref[to] = (jnp.dot(acc, wv_ref[...], preferred_element_type=_f32)
                      + bv_ref[...])


_EB = 2000


def _make_edge_body(last):
    def body(gr_ref, gc_ref, b_ref, wr_ref, w2_ref, b2_ref,
             x1_ref, bx1_ref, x2_ref, out_ref):
        grv = gr_ref[...]
        gcv = gc_ref[...]
        a = grv[:, 0:_D] + gcv[:, 0:_D]
        d = grv[:, _D:_D + 3] - gcv[:, _D:_D + 3]
        radial = jnp.sum(d * d, axis=1, keepdims=True)
        pre = a + radial * wr_ref[...] + b_ref[...]
        pre = pre * jax.nn.sigmoid(pre)
        m = jnp.dot(pre, w2_ref[...], preferred_element_type=_f32) + b2_ref[...]
        m = m * jax.nn.sigmoid(m)
        if last:
            t1 = (jnp.dot(m, x1_ref[...], preferred_element_type=_f32)
                  + bx1_ref[...])
            t1 = t1 * jax.nn.sigmoid(t1)
            tx = jnp.dot(t1, x2_ref[...], preferred_element_type=_f32)
            out_ref[...] = jnp.concatenate(
                [d * tx, jnp.ones((_EB, 1), _f32), jnp.zeros((_EB, 12), _f32)],
                axis=1)
        else:
            out_ref[...] = m
    return body


def _hupd_body(p_ref, hn_ref, wa_ref, wb_ref, b1_ref, w2_ref, b2_ref, out_ref):
    magg = p_ref[0] + p_ref[1]
    hv = hn_ref[...]
    u = (jnp.dot(hv, wa_ref[...], preferred_element_type=_f32)
         + jnp.dot(magg, wb_ref[...], preferred_element_type=_f32)
         + b1_ref[...])
    u = u * jax.nn.sigmoid(u)
    out_ref[...] = (hv + jnp.dot(u, w2_ref[...], preferred_element_type=_f32)
                    + b2_ref[...])


def _xupd_body(p_ref, xn_ref, vn_ref, sv_ref, out_ref):
    acc = p_ref[0] + p_ref[1]
    cnt = jnp.maximum(acc[:, 3:4], 1.0)
    agg = acc[:, 0:3] / cnt
    vout = sv_ref[...] * vn_ref[...] + agg
    out_ref[...] = xn_ref[...] + vout


def _make_gather_edge():
    total = _ET // _CH
    cpt = _BE // _CH
    niter = total // _NW
    out_h = jax.ShapeDtypeStruct((_ET, _GW), _f32)
    mesh = plsc.VectorSubcoreMesh(core_axis_name="c", subcore_axis_name="s")

    @functools.partial(
        pl.kernel, mesh=mesh,
        out_type=(out_h, out_h),
        compiler_params=pltpu.CompilerParams(use_tc_tiling_on_sc=False),
        scratch_types=[
            pltpu.VMEM((_CH,), jnp.int32),
            pltpu.VMEM((_CH, _GW), _f32),
            pltpu.SemaphoreType.DMA,
        ],
    )
    def k(tab_hr, tab_hc, idx_r, idx_c, o_hr, o_hc, idx_v, hrow_v, sem):
        wid = lax.axis_index("s") * 2 + lax.axis_index("c")

        def run_side(tab_h, idx, o_h):
            def body(kk, carry):
                g = wid + _NW * kk
                t = g // cpt
                j = g - t * cpt
                pltpu.sync_copy(idx.at[pl.ds(j * _CH, _CH)], idx_v)
                for i in range(_CH // 16):
                    idx_v[pl.ds(i * 16, 16)] = idx_v[pl.ds(i * 16, 16)] + t * _N
                pltpu.async_copy(tab_h.at[idx_v], hrow_v, sem).wait()
                pltpu.sync_copy(hrow_v, o_h.at[pl.ds(g * _CH, _CH)])
                return carry

            lax.fori_loop(0, niter, body, 0)

        run_side(tab_hr, idx_r, o_hr)
        run_side(tab_hc, idx_c, o_hc)

    return k


def _make_gather(n_tab, width, n_idx, t_rep, n_off):
    total = (n_idx * t_rep) // _CH
    cpt = n_idx // _CH
    niter = -(-total // _NW)
    out_sh = jax.ShapeDtypeStruct((n_idx * t_rep, width), _f32)
    mesh = plsc.VectorSubcoreMesh(core_axis_name="c", subcore_axis_name="s")

    @functools.partial(
        pl.kernel, mesh=mesh,
        out_type=(out_sh, out_sh),
        compiler_params=pltpu.CompilerParams(use_tc_tiling_on_sc=False),
        scratch_types=[
            pltpu.VMEM((_CH,), jnp.int32),
            pltpu.VMEM((_CH, width), _f32),
            pltpu.SemaphoreType.DMA,
        ],
    )
    def k(tab_r, tab_c, idx_r, idx_c, out_r, out_c, idx_v, row_v, sem):
        wid = lax.axis_index("s") * 2 + lax.axis_index("c")

        def run_side(tab, idx, out):
            def body(kk, carry):
                g = wid + _NW * kk

                @pl.when(g < total)
                def _():
                    t = g // cpt
                    j = g - t * cpt
                    pltpu.sync_copy(idx.at[pl.ds(j * _CH, _CH)], idx_v)
                    if t_rep > 1:
                        off = t * n_off
                        for i in range(_CH // 16):
                            idx_v[pl.ds(i * 16, 16)] = (
                                idx_v[pl.ds(i * 16, 16)] + off)
                    pltpu.async_copy(tab.at[idx_v], row_v, sem).wait()
                    pltpu.sync_copy(row_v, out.at[pl.ds(g * _CH, _CH)])
                return carry

            lax.fori_loop(0, niter, body, 0)

        run_side(tab_r, idx_r, out_r)
        run_side(tab_c, idx_c, out_c)

    return k


_SEG = 9216
_SROWS = _SEG // 16


def _make_scatter(width):
    total = _ET // _CH
    cpt = _BE // _CH
    niter = -(-total // _NW)
    rows = _SROWS
    mesh = plsc.VectorSubcoreMesh(core_axis_name="c", subcore_axis_name="s")

    @functools.partial(
        pl.kernel, mesh=mesh,
        out_type=jax.ShapeDtypeStruct((2 * _SEG, width), _f32),
        compiler_params=pltpu.CompilerParams(use_tc_tiling_on_sc=False),
        scratch_types=[
            pltpu.VMEM((_CH,), jnp.int32),
            pltpu.VMEM((_CH, width), _f32),
            pltpu.VMEM((rows, width), _f32),
            pltpu.VMEM_SHARED((_SEG, width), _f32),
        ],
    )
    def k(val_hbm, idx_hbm, zero_hbm, out_hbm, idx_v, val_v, blk_v, shared):
        cid = lax.axis_index("c")
        sid = lax.axis_index("s")
        wid = sid * 2 + cid
        pltpu.sync_copy(zero_hbm, shared.at[pl.ds(sid * rows, rows)])
        plsc.subcore_barrier()

        def body(kk, carry):
            g = wid + _NW * kk

            @pl.when(g < total)
            def _():
                t = g // cpt
                j = g - t * cpt
                pltpu.sync_copy(idx_hbm.at[pl.ds(j * _CH, _CH)], idx_v)
                off = t * _N
                for i in range(_CH // 16):
                    idx_v[pl.ds(i * 16, 16)] = idx_v[pl.ds(i * 16, 16)] + off
                pltpu.sync_copy(val_hbm.at[pl.ds(g * _CH, _CH)], val_v)
                pltpu.sync_copy(val_v, shared.at[idx_v], add=True)
            return carry

        lax.fori_loop(0, niter, body, 0)
        plsc.subcore_barrier()
        pltpu.sync_copy(shared.at[pl.ds(sid * rows, rows)], blk_v)
        pltpu.sync_copy(blk_v, out_hbm.at[pl.ds(cid * _SEG + sid * rows, rows)])

    return k


_make_gather = functools.lru_cache(maxsize=None)(_make_gather)
_make_gather_edge = functools.lru_cache(maxsize=None)(_make_gather_edge)
_make_scatter = functools.lru_cache(maxsize=None)(_make_scatter)


def _gather_loc(*args):
    return _make_gather(_BN, 16, _BE, 1, 0)(*args)


def _gather_edge(*args):
    return _make_gather_edge()(*args)


def _scatter_m(*args):
    return _make_scatter(_D)(*args)


def _scatter_x(*args):
    return _make_scatter(16)(*args)


def _timestep_embedding(num_timesteps, dim):
    half = dim // 2
    emb = math.log(10000.0) / (half - 1)
    freqs = jnp.exp(jnp.arange(half, dtype=_f32) * -emb)
    te = jnp.arange(num_timesteps, dtype=_f32)[:, None] * freqs[None, :]
    return jnp.concatenate([jnp.sin(te), jnp.cos(te)], axis=-1)


def _fold_time_conv(wr, wi):
    k = jnp.arange(_MODES, dtype=_f32)
    t = jnp.arange(_T, dtype=_f32)
    w_k = jnp.where(k == 0, 1.0, 2.0)
    ang = (2.0 * math.pi / _T) * k[None, None, :] * (
        t[None, :, None] - t[:, None, None])
    m = (jnp.einsum('itk,k,kco->itco', jnp.cos(ang), w_k, wr)
         - jnp.einsum('itk,k,kco->itco', jnp.sin(ang), w_k, wi)) / _T
    eye = jnp.eye(wr.shape[1], dtype=_f32)
    return m + jnp.eye(_T, dtype=_f32)[:, :, None, None] * eye[None, None]


def kernel(x_0, v_0, concatenated_features, edge_attr, source_node_indices,
           target_node_indices, params):
    P = params['layers']

    te = _timestep_embedding(_T, _TIME_EMB)
    time_emb = jnp.reshape(
        jnp.broadcast_to(te[:, None, :], (_T, _BN, _TIME_EMB)),
        (_BTN, _TIME_EMB))
    h2 = jnp.reshape(concatenated_features[..., -2:], (_BTN, 2))
    feat = jnp.concatenate([h2, time_emb], axis=1)
    x = jnp.reshape(x_0[..., :3], (_BTN, 3))
    v = jnp.reshape(v_0[..., :3], (_BTN, 3))
    loc = jnp.reshape(x_0[:, 0, :, :3], (_BN, 3))
    locpad = jnp.concatenate([loc, jnp.zeros((_BN, 13), _f32)], axis=1)
    rows0 = jnp.reshape(source_node_indices, (_BE,)).astype(jnp.int32)
    cols0 = jnp.reshape(target_node_indices, (_BE,)).astype(jnp.int32)
    ea = jnp.reshape(edge_attr, (_BE, 2))

    mh = [_fold_time_conv(P[l]['wt_r'], P[l]['wt_i']) for l in range(2)]
    mx = [_fold_time_conv(P[l]['wtx_r'], P[l]['wtx_i']) for l in range(2)]
    wea_s = jnp.stack([P[l]['We1'][129:131] for l in range(2)])
    wld_s = jnp.stack([P[l]['We1'][131] for l in range(2)])
    be1_s = jnp.stack([P[l]['be1'] for l in range(2)])
    z64 = jnp.zeros((_SROWS, _D), _f32)
    z16 = jnp.zeros((_SROWS, 16), _f32)

    x0p = jnp.reshape(jnp.transpose(x_0[..., :3], (2, 0, 1, 3)), (_N, 48))
    h, lm48 = pl.pallas_call(
        _prep_body,
        out_shape=(jax.ShapeDtypeStruct((_BTN, _D), _f32),
                   jax.ShapeDtypeStruct((1, 48), _f32)),
    )(feat, params['W_emb'], params['b_emb'][None, :], x0p)
    lm = jnp.reshape(
        jnp.broadcast_to(jnp.reshape(lm48, (_B * _T, 1, 3)), (_B * _T, _N, 3)),
        (_BTN, 3))

    gl_r, gl_c = _gather_loc(locpad, locpad, rows0, cols0)

    bias = pl.pallas_call(
        _bias_body,
        grid=(8,),
        in_specs=[
            pl.BlockSpec((_BE // 8, 16), lambda i: (i, 0)),
            pl.BlockSpec((_BE // 8, 16), lambda i: (i, 0)),
            pl.BlockSpec((_BE // 8, 2), lambda i: (i, 0)),
            pl.BlockSpec((2, 2, _D), lambda i: (0, 0, 0)),
            pl.BlockSpec((2, _D), lambda i: (0, 0)),
            pl.BlockSpec((2, _D), lambda i: (0, 0)),
        ],
        out_specs=pl.BlockSpec((2, _BE // 8, _D), lambda i: (0, i, 0)),
        out_shape=jax.ShapeDtypeStruct((2, _BE, _D), _f32),
    )(gl_r, gl_c, ea, wea_s, wld_s, be1_s)

    x_out = None
    for l in range(2):
        Pl = P[l]
        last = l == 1
        k1_out = pl.pallas_call(
            _k1_body,
            grid=(_BN // _K1_BLK,),
            in_specs=[
                pl.BlockSpec((_T, _K1_BLK, _D), lambda i: (0, i, 0)),
                pl.BlockSpec((_T, _K1_BLK, 3), lambda i: (0, i, 0)),
                pl.BlockSpec((_T, _K1_BLK, 3), lambda i: (0, i, 0)),
                pl.BlockSpec((_T, _K1_BLK, 3), lambda i: (0, i, 0)),
                pl.BlockSpec((_T, _T, _D, _D), lambda i: (0, 0, 0, 0)),
                pl.BlockSpec((_T, _T, 2, 2), lambda i: (0, 0, 0, 0)),
                pl.BlockSpec((_D, _D), lambda i: (0, 0)),
                pl.BlockSpec((_D, _D), lambda i: (0, 0)),
                pl.BlockSpec((_D, 1), lambda i: (0, 0)),
                pl.BlockSpec((1, 1), lambda i: (0, 0)),
            ],
            out_specs=[
                pl.BlockSpec((_T, _K1_BLK, _D), lambda i: (0, i, 0)),
                pl.BlockSpec((_T, _K1_BLK, _GW), lambda i: (0, i, 0)),
                pl.BlockSpec((_T, _K1_BLK, _GW), lambda i: (0, i, 0)),
                pl.BlockSpec((_T, _K1_BLK, 3), lambda i: (0, i, 0)),
                pl.BlockSpec((_T, _K1_BLK, 3), lambda i: (0, i, 0)),
                pl.BlockSpec((_T, _K1_BLK, 1), lambda i: (0, i, 0)),
            ],
            out_shape=(
                jax.ShapeDtypeStruct((_T, _BN, _D), _f32),
                jax.ShapeDtypeStruct((_T, _BN, _GW), _f32),
                jax.ShapeDtypeStruct((_T, _BN, _GW), _f32),
                jax.ShapeDtypeStruct((_T, _BN, 3), _f32),
                jax.ShapeDtypeStruct((_T, _BN, 3), _f32),
                jax.ShapeDtypeStruct((_T, _BN, 1), _f32),
            ),
        )(jnp.reshape(h, (_T, _BN, _D)), jnp.reshape(x, (_T, _BN, 3)),
          jnp.reshape(v, (_T, _BN, 3)), jnp.reshape(lm, (_T, _BN, 3)),
          mh[l], mx[l], Pl['We1'][0:_D], Pl['We1'][_D:2 * _D], Pl['Wv'],
          Pl['bv'][None, :])
        hn_t, gr_t, gc_t, xn_t, vn_t, sv_t = k1_out
        hn = jnp.reshape(hn_t, (_BTN, _D))
        tab_hr = jnp.reshape(gr_t, (_BTN, _GW))
        tab_hc = jnp.reshape(gc_t, (_BTN, _GW))
        xn = jnp.reshape(xn_t, (_BTN, 3))
        vn = jnp.reshape(vn_t, (_BTN, 3))

        g_hr, g_hc = _gather_edge(tab_hr, tab_hc, rows0, cols0)

        w_out = 16 if last else _D
        val = pl.pallas_call(
            _make_edge_body(last),
            grid=(_ET // _EB,),
            in_specs=[
                pl.BlockSpec((_EB, _GW), lambda i: (i, 0)),
                pl.BlockSpec((_EB, _GW), lambda i: (i, 0)),
                pl.BlockSpec((_EB, _D), lambda i: (i % (_BE // _EB), 0)),
                pl.BlockSpec((1, _D), lambda i: (0, 0)),
                pl.BlockSpec((_D, _D), lambda i: (0, 0)),
                pl.BlockSpec((1, _D), lambda i: (0, 0)),
                pl.BlockSpec((_D, _D), lambda i: (0, 0)),
                pl.BlockSpec((1, _D), lambda i: (0, 0)),
                pl.BlockSpec((_D, 1), lambda i: (0, 0)),
            ],
            out_specs=pl.BlockSpec((_EB, w_out), lambda i: (i, 0)),
            out_shape=jax.ShapeDtypeStruct((_ET, w_out), _f32),
        )(g_hr, g_hc, bias[l], Pl['We1'][128][None, :], Pl['We2'],
          Pl['be2'][None, :], Pl['Wx1'], Pl['bx1'][None, :], Pl['Wx2'])

        if not last:
            part = _scatter_m(val, rows0, z64)
            p = jnp.concatenate(
                [jnp.reshape(part, (2, _SEG, _D)),
                 jnp.zeros((2, _BTN - _SEG, _D), _f32)], axis=1)
            h = pl.pallas_call(
                _hupd_body,
                grid=(8,),
                in_specs=[
                    pl.BlockSpec((2, _BTN // 8, _D), lambda i: (0, i, 0)),
                    pl.BlockSpec((_BTN // 8, _D), lambda i: (i, 0)),
                    pl.BlockSpec((_D, _D), lambda i: (0, 0)),
                    pl.BlockSpec((_D, _D), lambda i: (0, 0)),
                    pl.BlockSpec((1, _D), lambda i: (0, 0)),
                    pl.BlockSpec((_D, _D), lambda i: (0, 0)),
                    pl.BlockSpec((1, _D), lambda i: (0, 0)),
                ],
                out_specs=pl.BlockSpec((_BTN // 8, _D), lambda i: (i, 0)),
                out_shape=jax.ShapeDtypeStruct((_BTN, _D), _f32),
            )(p, hn, Pl['Wh1'][0:_D], Pl['Wh1'][_D:2 * _D],
              Pl['bh1'][None, :], Pl['Wh2'], Pl['bh2'][None, :])
            x, v = xn, vn
        else:
            part = _scatter_x(val, rows0, z16)
            p = jnp.concatenate(
                [jnp.reshape(part, (2, _SEG, 16)),
                 jnp.zeros((2, _BTN - _SEG, 16), _f32)], axis=1)
            x_out = pl.pallas_call(
                _xupd_body,
                grid=(8,),
                in_specs=[
                    pl.BlockSpec((2, _BTN // 8, 16), lambda i: (0, i, 0)),
                    pl.BlockSpec((_BTN // 8, 3), lambda i: (i, 0)),
                    pl.BlockSpec((_BTN // 8, 3), lambda i: (i, 0)),
                    pl.BlockSpec((_BTN // 8, 1), lambda i: (i, 0)),
                ],
                out_specs=pl.BlockSpec((_BTN // 8, 3), lambda i: (i, 0)),
                out_shape=jax.ShapeDtypeStruct((_BTN, 3), _f32),
            )(p, xn, vn, jnp.reshape(sv_t, (_BTN, 1)))

    return jnp.reshape(x_out, (_B, _T, _N, 3))


# speedup vs baseline: 1.2767x; 1.0220x over previous
import functools
import math

import jax
import jax.numpy as jnp
from jax import lax
from jax.experimental import pallas as pl
from jax.experimental.pallas import tpu as pltpu
from jax.experimental.pallas import tpu_sc as plsc

_B, _T, _N, _E = 2, 8, 1000, 16000
_D = 64
_MODES = 4
_TIME_EMB = 32
_BN = _B * _N
_BTN = _B * _T * _N
_BE = _B * _E
_ET = _T * _BE
_GW = 80
_XW = 16
_CH = 800
_NW = 32

_f32 = jnp.float32


def _prep_body(feat_ref, wemb_ref, bemb_ref, x0_ref, h_ref, lm_ref):
    h_ref[...] = (
        jnp.dot(feat_ref[...], wemb_ref[...], preferred_element_type=_f32)
        + bemb_ref[...]
    )
    lm_ref[...] = jnp.mean(x0_ref[...], axis=0, keepdims=True)


def _bias_body(glr_ref, glc_ref, ea_ref, wea_ref, wld_ref, be1_ref, out_ref):
    d = glr_ref[:, 0:3] - glc_ref[:, 0:3]
    ld = jnp.sum(d * d, axis=1, keepdims=True)
    eav = ea_ref[...]
    for l in range(2):
        out_ref[l] = (
            jnp.dot(eav, wea_ref[l], preferred_element_type=_f32)
            + ld * wld_ref[l]
            + be1_ref[l]
        )


_K1_BLK = 400


def _k1_body(h_ref, x_ref, v_ref, lm_ref, mh_ref, mx_ref, wa_ref, wb_ref,
             wv_ref, bv_ref, hn_ref, gr_ref, gc_ref, xn_ref, vn_ref,
             sv_ref):
    hin = h_ref[...]
    mhv = mh_ref[...]
    mxv = mx_ref[...]
    xc = x_ref[...] - lm_ref[...]
    vv = v_ref[...]
    pad = jnp.zeros((_K1_BLK, _GW - _D - 3), _f32)
    for to in range(_T):
        acc = jnp.zeros((_K1_BLK, _D), _f32)
        for ti in range(_T):
            acc = acc + jnp.dot(hin[ti], mhv[ti, to],
                                preferred_element_type=_f32)
        hn_ref[to] = acc
        ax = jnp.zeros((_K1_BLK, 3), _f32)
        av = jnp.zeros((_K1_BLK, 3), _f32)
        for ti in range(_T):
            ax = ax + xc[ti] * mxv[ti, to, 0, 0] + vv[ti] * mxv[ti, to, 1, 0]
            av = av + xc[ti] * mxv[ti, to, 0, 1] + vv[ti] * mxv[ti, to, 1, 1]
        xo = ax + lm_ref[to]
        xn_ref[to] = xo
        vn_ref[to] = av
        gr_ref[to] = jnp.concatenate(
            [jnp.dot(acc, wa_ref[...], preferred_element_type=_f32), xo, pad],
            axis=1)
        gc_ref[to] = jnp.concatenate(
            [jnp.dot(acc, wb_ref[...], preferred_element_type=_f32), xo, pad],
            axis=1)
        sv_ref[to] = (jnp.dot(acc, wv_ref[...], preferred_element_type=_f32)
                      + bv_ref[...])


_EB = 2000


def _make_edge_body(last):
    def body(gr_ref, gc_ref, b_ref, wr_ref, w2_ref, b2_ref,
             x1_ref, bx1_ref, x2_ref, out_ref):
        grv = gr_ref[...]
        gcv = gc_ref[...]
        a = grv[:, 0:_D] + gcv[:, 0:_D]
        d = grv[:, _D:_D + 3] - gcv[:, _D:_D + 3]
        radial = jnp.sum(d * d, axis=1, keepdims=True)
        pre = a + radial * wr_ref[...] + b_ref[...]
        pre = pre * jax.nn.sigmoid(pre)
        m = jnp.dot(pre, w2_ref[...], preferred_element_type=_f32) + b2_ref[...]
        m = m * jax.nn.sigmoid(m)
        if last:
            t1 = (jnp.dot(m, x1_ref[...], preferred_element_type=_f32)
                  + bx1_ref[...])
            t1 = t1 * jax.nn.sigmoid(t1)
            tx = jnp.dot(t1, x2_ref[...], preferred_element_type=_f32)
            out_ref[...] = jnp.concatenate(
                [d * tx, jnp.ones((_EB, 1), _f32), jnp.zeros((_EB, 12), _f32)],
                axis=1)
        else:
            out_ref[...] = m
    return body


def _hupd_body(p_ref, hn_ref, wa_ref, wb_ref, b1_ref, w2_ref, b2_ref, out_ref):
    magg = p_ref[0] + p_ref[1]
    hv = hn_ref[...]
    u = (jnp.dot(hv, wa_ref[...], preferred_element_type=_f32)
         + jnp.dot(magg, wb_ref[...], preferred_element_type=_f32)
         + b1_ref[...])
    u = u * jax.nn.sigmoid(u)
    out_ref[...] = (hv + jnp.dot(u, w2_ref[...], preferred_element_type=_f32)
                    + b2_ref[...])


def _xupd_body(p_ref, xn_ref, vn_ref, sv_ref, out_ref):
    acc = p_ref[0] + p_ref[1]
    cnt = jnp.maximum(acc[:, 3:4], 1.0)
    agg = acc[:, 0:3] / cnt
    vout = sv_ref[...] * vn_ref[...] + agg
    out_ref[...] = xn_ref[...] + vout


_CHG = 640


def _make_gather_edge():
    total = _ET // _CHG
    cpt = _BE // _CHG
    niter = -(-total // _NW)
    out_h = jax.ShapeDtypeStruct((_ET, _GW), _f32)
    mesh = plsc.VectorSubcoreMesh(core_axis_name="c", subcore_axis_name="s")

    @functools.partial(
        pl.kernel, mesh=mesh,
        out_type=(out_h, out_h),
        compiler_params=pltpu.CompilerParams(use_tc_tiling_on_sc=False),
        scratch_types=[
            pltpu.VMEM((_CHG,), jnp.int32),
            pltpu.VMEM((_CHG,), jnp.int32),
            pltpu.VMEM((_CHG, _GW), _f32),
            pltpu.VMEM((_CHG, _GW), _f32),
            pltpu.SemaphoreType.DMA,
            pltpu.SemaphoreType.DMA,
        ],
    )
    def k(tab_hr, tab_hc, idx_r, idx_c, o_hr, o_hc,
          idx_vr, idx_vc, row_r, row_c, sem_r, sem_c):
        wid = lax.axis_index("s") * 2 + lax.axis_index("c")

        def body(kk, carry):
            g = wid + _NW * kk

            @pl.when(g < total)
            def _():
                t = g // cpt
                j = g - t * cpt
                off = t * _N
                pltpu.sync_copy(idx_r.at[pl.ds(j * _CHG, _CHG)], idx_vr)
                for i in range(_CHG // 16):
                    idx_vr[pl.ds(i * 16, 16)] = idx_vr[pl.ds(i * 16, 16)] + off
                d_r = pltpu.async_copy(tab_hr.at[idx_vr], row_r, sem_r)
                pltpu.sync_copy(idx_c.at[pl.ds(j * _CHG, _CHG)], idx_vc)
                for i in range(_CHG // 16):
                    idx_vc[pl.ds(i * 16, 16)] = idx_vc[pl.ds(i * 16, 16)] + off
                d_c = pltpu.async_copy(tab_hc.at[idx_vc], row_c, sem_c)
                d_r.wait()
                pltpu.sync_copy(row_r, o_hr.at[pl.ds(g * _CHG, _CHG)])
                d_c.wait()
                pltpu.sync_copy(row_c, o_hc.at[pl.ds(g * _CHG, _CHG)])
            return carry

        lax.fori_loop(0, niter, body, 0)

    return k


def _make_gather(n_tab, width, n_idx, t_rep, n_off):
    total = (n_idx * t_rep) // _CH
    cpt = n_idx // _CH
    niter = -(-total // _NW)
    out_sh = jax.ShapeDtypeStruct((n_idx * t_rep, width), _f32)
    mesh = plsc.VectorSubcoreMesh(core_axis_name="c", subcore_axis_name="s")

    @functools.partial(
        pl.kernel, mesh=mesh,
        out_type=(out_sh, out_sh),
        compiler_params=pltpu.CompilerParams(use_tc_tiling_on_sc=False),
        scratch_types=[
            pltpu.VMEM((_CH,), jnp.int32),
            pltpu.VMEM((_CH, width), _f32),
            pltpu.SemaphoreType.DMA,
        ],
    )
    def k(tab_r, tab_c, idx_r, idx_c, out_r, out_c, idx_v, row_v, sem):
        wid = lax.axis_index("s") * 2 + lax.axis_index("c")

        def run_side(tab, idx, out):
            def body(kk, carry):
                g = wid + _NW * kk

                @pl.when(g < total)
                def _():
                    t = g // cpt
                    j = g - t * cpt
                    pltpu.sync_copy(idx.at[pl.ds(j * _CH, _CH)], idx_v)
                    if t_rep > 1:
                        off = t * n_off
                        for i in range(_CH // 16):
                            idx_v[pl.ds(i * 16, 16)] = (
                                idx_v[pl.ds(i * 16, 16)] + off)
                    pltpu.async_copy(tab.at[idx_v], row_v, sem).wait()
                    pltpu.sync_copy(row_v, out.at[pl.ds(g * _CH, _CH)])
                return carry

            lax.fori_loop(0, niter, body, 0)

        run_side(tab_r, idx_r, out_r)
        run_side(tab_c, idx_c, out_c)

    return k


_SEG = 9216
_SROWS = _SEG // 16


def _make_scatter(width):
    total = _ET // _CH
    cpt = _BE // _CH
    niter = -(-total // _NW)
    rows = _SROWS
    mesh = plsc.VectorSubcoreMesh(core_axis_name="c", subcore_axis_name="s")

    @functools.partial(
        pl.kernel, mesh=mesh,
        out_type=jax.ShapeDtypeStruct((2 * _SEG, width), _f32),
        compiler_params=pltpu.CompilerParams(use_tc_tiling_on_sc=False),
        scratch_types=[
            pltpu.VMEM((_CH,), jnp.int32),
            pltpu.VMEM((_CH, width), _f32),
            pltpu.VMEM((rows, width), _f32),
            pltpu.VMEM_SHARED((_SEG, width), _f32),
        ],
    )
    def k(val_hbm, idx_hbm, zero_hbm, out_hbm, idx_v, val_v, blk_v, shared):
        cid = lax.axis_index("c")
        sid = lax.axis_index("s")
        wid = sid * 2 + cid
        pltpu.sync_copy(zero_hbm, shared.at[pl.ds(sid * rows, rows)])
        plsc.subcore_barrier()

        def body(kk, carry):
            g = wid + _NW * kk

            @pl.when(g < total)
            def _():
                t = g // cpt
                j = g - t * cpt
                pltpu.sync_copy(idx_hbm.at[pl.ds(j * _CH, _CH)], idx_v)
                off = t * _N
                for i in range(_CH // 16):
                    idx_v[pl.ds(i * 16, 16)] = idx_v[pl.ds(i * 16, 16)] + off
                pltpu.sync_copy(val_hbm.at[pl.ds(g * _CH, _CH)], val_v)
                pltpu.sync_copy(val_v, shared.at[idx_v], add=True)
            return carry

        lax.fori_loop(0, niter, body, 0)
        plsc.subcore_barrier()
        pltpu.sync_copy(shared.at[pl.ds(sid * rows, rows)], blk_v)
        pltpu.sync_copy(blk_v, out_hbm.at[pl.ds(cid * _SEG + sid * rows, rows)])

    return k


_make_gather = functools.lru_cache(maxsize=None)(_make_gather)
_make_gather_edge = functools.lru_cache(maxsize=None)(_make_gather_edge)
_make_scatter = functools.lru_cache(maxsize=None)(_make_scatter)


def _gather_loc(*args):
    return _make_gather(_BN, 16, _BE, 1, 0)(*args)


def _gather_edge(*args):
    return _make_gather_edge()(*args)


def _scatter_m(*args):
    return _make_scatter(_D)(*args)


def _scatter_x(*args):
    return _make_scatter(16)(*args)


def _timestep_embedding(num_timesteps, dim):
    half = dim // 2
    emb = math.log(10000.0) / (half - 1)
    freqs = jnp.exp(jnp.arange(half, dtype=_f32) * -emb)
    te = jnp.arange(num_timesteps, dtype=_f32)[:, None] * freqs[None, :]
    return jnp.concatenate([jnp.sin(te), jnp.cos(te)], axis=-1)


def _fold_time_conv(wr, wi):
    k = jnp.arange(_MODES, dtype=_f32)
    t = jnp.arange(_T, dtype=_f32)
    w_k = jnp.where(k == 0, 1.0, 2.0)
    ang = (2.0 * math.pi / _T) * k[None, None, :] * (
        t[None, :, None] - t[:, None, None])
    m = (jnp.einsum('itk,k,kco->itco', jnp.cos(ang), w_k, wr)
         - jnp.einsum('itk,k,kco->itco', jnp.sin(ang), w_k, wi)) / _T
    eye = jnp.eye(wr.shape[1], dtype=_f32)
    return m + jnp.eye(_T, dtype=_f32)[:, :, None, None] * eye[None, None]


def kernel(x_0, v_0, concatenated_features, edge_attr, source_node_indices,
           target_node_indices, params):
    P = params['layers']

    te = _timestep_embedding(_T, _TIME_EMB)
    time_emb = jnp.reshape(
        jnp.broadcast_to(te[:, None, :], (_T, _BN, _TIME_EMB)),
        (_BTN, _TIME_EMB))
    h2 = jnp.reshape(concatenated_features[..., -2:], (_BTN, 2))
    feat = jnp.concatenate([h2, time_emb], axis=1)
    x = jnp.reshape(x_0[..., :3], (_BTN, 3))
    v = jnp.reshape(v_0[..., :3], (_BTN, 3))
    loc = jnp.reshape(x_0[:, 0, :, :3], (_BN, 3))
    locpad = jnp.concatenate([loc, jnp.zeros((_BN, 13), _f32)], axis=1)
    rows0 = jnp.reshape(source_node_indices, (_BE,)).astype(jnp.int32)
    cols0 = jnp.reshape(target_node_indices, (_BE,)).astype(jnp.int32)
    ea = jnp.reshape(edge_attr, (_BE, 2))

    mh = [_fold_time_conv(P[l]['wt_r'], P[l]['wt_i']) for l in range(2)]
    mx = [_fold_time_conv(P[l]['wtx_r'], P[l]['wtx_i']) for l in range(2)]
    wea_s = jnp.stack([P[l]['We1'][129:131] for l in range(2)])
    wld_s = jnp.stack([P[l]['We1'][131] for l in range(2)])
    be1_s = jnp.stack([P[l]['be1'] for l in range(2)])
    z64 = jnp.zeros((_SROWS, _D), _f32)
    z16 = jnp.zeros((_SROWS, 16), _f32)

    x0p = jnp.reshape(jnp.transpose(x_0[..., :3], (2, 0, 1, 3)), (_N, 48))
    h, lm48 = pl.pallas_call(
        _prep_body,
        out_shape=(jax.ShapeDtypeStruct((_BTN, _D), _f32),
                   jax.ShapeDtypeStruct((1, 48), _f32)),
    )(feat, params['W_emb'], params['b_emb'][None, :], x0p)
    lm = jnp.reshape(
        jnp.broadcast_to(jnp.reshape(lm48, (_B * _T, 1, 3)), (_B * _T, _N, 3)),
        (_BTN, 3))

    gl_r, gl_c = _gather_loc(locpad, locpad, rows0, cols0)

    bias = pl.pallas_call(
        _bias_body,
        grid=(8,),
        in_specs=[
            pl.BlockSpec((_BE // 8, 16), lambda i: (i, 0)),
            pl.BlockSpec((_BE // 8, 16), lambda i: (i, 0)),
            pl.BlockSpec((_BE // 8, 2), lambda i: (i, 0)),
            pl.BlockSpec((2, 2, _D), lambda i: (0, 0, 0)),
            pl.BlockSpec((2, _D), lambda i: (0, 0)),
            pl.BlockSpec((2, _D), lambda i: (0, 0)),
        ],
        out_specs=pl.BlockSpec((2, _BE // 8, _D), lambda i: (0, i, 0)),
        out_shape=jax.ShapeDtypeStruct((2, _BE, _D), _f32),
    )(gl_r, gl_c, ea, wea_s, wld_s, be1_s)

    x_out = None
    for l in range(2):
        Pl = P[l]
        last = l == 1
        k1_out = pl.pallas_call(
            _k1_body,
            grid=(_BN // _K1_BLK,),
            in_specs=[
                pl.BlockSpec((_T, _K1_BLK, _D), lambda i: (0, i, 0)),
                pl.BlockSpec((_T, _K1_BLK, 3), lambda i: (0, i, 0)),
                pl.BlockSpec((_T, _K1_BLK, 3), lambda i: (0, i, 0)),
                pl.BlockSpec((_T, _K1_BLK, 3), lambda i: (0, i, 0)),
                pl.BlockSpec((_T, _T, _D, _D), lambda i: (0, 0, 0, 0)),
                pl.BlockSpec((_T, _T, 2, 2), lambda i: (0, 0, 0, 0)),
                pl.BlockSpec((_D, _D), lambda i: (0, 0)),
                pl.BlockSpec((_D, _D), lambda i: (0, 0)),
                pl.BlockSpec((_D, 1), lambda i: (0, 0)),
                pl.BlockSpec((1, 1), lambda i: (0, 0)),
            ],
            out_specs=[
                pl.BlockSpec((_T, _K1_BLK, _D), lambda i: (0, i, 0)),
                pl.BlockSpec((_T, _K1_BLK, _GW), lambda i: (0, i, 0)),
                pl.BlockSpec((_T, _K1_BLK, _GW), lambda i: (0, i, 0)),
                pl.BlockSpec((_T, _K1_BLK, 3), lambda i: (0, i, 0)),
                pl.BlockSpec((_T, _K1_BLK, 3), lambda i: (0, i, 0)),
                pl.BlockSpec((_T, _K1_BLK, 1), lambda i: (0, i, 0)),
            ],
            out_shape=(
                jax.ShapeDtypeStruct((_T, _BN, _D), _f32),
                jax.ShapeDtypeStruct((_T, _BN, _GW), _f32),
                jax.ShapeDtypeStruct((_T, _BN, _GW), _f32),
                jax.ShapeDtypeStruct((_T, _BN, 3), _f32),
                jax.ShapeDtypeStruct((_T, _BN, 3), _f32),
                jax.ShapeDtypeStruct((_T, _BN, 1), _f32),
            ),
        )(jnp.reshape(h, (_T, _BN, _D)), jnp.reshape(x, (_T, _BN, 3)),
          jnp.reshape(v, (_T, _BN, 3)), jnp.reshape(lm, (_T, _BN, 3)),
          mh[l], mx[l], Pl['We1'][0:_D], Pl['We1'][_D:2 * _D], Pl['Wv'],
          Pl['bv'][None, :])
        hn_t, gr_t, gc_t, xn_t, vn_t, sv_t = k1_out
        hn = jnp.reshape(hn_t, (_BTN, _D))
        tab_hr = jnp.reshape(gr_t, (_BTN, _GW))
        tab_hc = jnp.reshape(gc_t, (_BTN, _GW))
        xn = jnp.reshape(xn_t, (_BTN, 3))
        vn = jnp.reshape(vn_t, (_BTN, 3))

        g_hr, g_hc = _gather_edge(tab_hr, tab_hc, rows0, cols0)

        w_out = 16 if last else _D
        val = pl.pallas_call(
            _make_edge_body(last),
            grid=(_ET // _EB,),
            in_specs=[
                pl.BlockSpec((_EB, _GW), lambda i: (i, 0)),
                pl.BlockSpec((_EB, _GW), lambda i: (i, 0)),
                pl.BlockSpec((_EB, _D), lambda i: (i % (_BE // _EB), 0)),
                pl.BlockSpec((1, _D), lambda i: (0, 0)),
                pl.BlockSpec((_D, _D), lambda i: (0, 0)),
                pl.BlockSpec((1, _D), lambda i: (0, 0)),
                pl.BlockSpec((_D, _D), lambda i: (0, 0)),
                pl.BlockSpec((1, _D), lambda i: (0, 0)),
                pl.BlockSpec((_D, 1), lambda i: (0, 0)),
            ],
            out_specs=pl.BlockSpec((_EB, w_out), lambda i: (i, 0)),
            out_shape=jax.ShapeDtypeStruct((_ET, w_out), _f32),
        )(g_hr, g_hc, bias[l], Pl['We1'][128][None, :], Pl['We2'],
          Pl['be2'][None, :], Pl['Wx1'], Pl['bx1'][None, :], Pl['Wx2'])

        if not last:
            part = _scatter_m(val, rows0, z64)
            p = jnp.concatenate(
                [jnp.reshape(part, (2, _SEG, _D)),
                 jnp.zeros((2, _BTN - _SEG, _D), _f32)], axis=1)
            h = pl.pallas_call(
                _hupd_body,
                grid=(8,),
                in_specs=[
                    pl.BlockSpec((2, _BTN // 8, _D), lambda i: (0, i, 0)),
                    pl.BlockSpec((_BTN // 8, _D), lambda i: (i, 0)),
                    pl.BlockSpec((_D, _D), lambda i: (0, 0)),
                    pl.BlockSpec((_D, _D), lambda i: (0, 0)),
                    pl.BlockSpec((1, _D), lambda i: (0, 0)),
                    pl.BlockSpec((_D, _D), lambda i: (0, 0)),
                    pl.BlockSpec((1, _D), lambda i: (0, 0)),
                ],
                out_specs=pl.BlockSpec((_BTN // 8, _D), lambda i: (i, 0)),
                out_shape=jax.ShapeDtypeStruct((_BTN, _D), _f32),
            )(p, hn, Pl['Wh1'][0:_D], Pl['Wh1'][_D:2 * _D],
              Pl['bh1'][None, :], Pl['Wh2'], Pl['bh2'][None, :])
            x, v = xn, vn
        else:
            part = _scatter_x(val, rows0, z16)
            p = jnp.concatenate(
                [jnp.reshape(part, (2, _SEG, 16)),
                 jnp.zeros((2, _BTN - _SEG, 16), _f32)], axis=1)
            x_out = pl.pallas_call(
                _xupd_body,
                grid=(8,),
                in_specs=[
                    pl.BlockSpec((2, _BTN // 8, 16), lambda i: (0, i, 0)),
                    pl.BlockSpec((_BTN // 8, 3), lambda i: (i, 0)),
                    pl.BlockSpec((_BTN // 8, 3), lambda i: (i, 0)),
                    pl.BlockSpec((_BTN // 8, 1), lambda i: (i, 0)),
                ],
                out_specs=pl.BlockSpec((_BTN // 8, 3), lambda i: (i, 0)),
                out_shape=jax.ShapeDtypeStruct((_BTN, 3), _f32),
            )(p, xn, vn, jnp.reshape(sv_t, (_BTN, 1)))

    return jnp.reshape(x_out, (_B, _T, _N, 3))


# speedup vs baseline: 1.2835x; 1.0053x over previous
import functools
import math

import jax
import jax.numpy as jnp
from jax import lax
from jax.experimental import pallas as pl
from jax.experimental.pallas import tpu as pltpu
from jax.experimental.pallas import tpu_sc as plsc

_B, _T, _N, _E = 2, 8, 1000, 16000
_D = 64
_MODES = 4
_TIME_EMB = 32
_BN = _B * _N
_BTN = _B * _T * _N
_BE = _B * _E
_ET = _T * _BE
_GW = 80
_XW = 16
_CH = 800
_NW = 32

_f32 = jnp.float32


def _prep_body(feat_ref, wemb_ref, bemb_ref, x0_ref, h_ref, lm_ref):
    h_ref[...] = (
        jnp.dot(feat_ref[...], wemb_ref[...], preferred_element_type=_f32)
        + bemb_ref[...]
    )
    lm_ref[...] = jnp.mean(x0_ref[...], axis=0, keepdims=True)


def _bias_body(glr_ref, glc_ref, ea_ref, wea_ref, wld_ref, be1_ref, out_ref):
    d = glr_ref[:, 0:3] - glc_ref[:, 0:3]
    ld = jnp.sum(d * d, axis=1, keepdims=True)
    eav = ea_ref[...]
    for l in range(2):
        out_ref[l] = (
            jnp.dot(eav, wea_ref[l], preferred_element_type=_f32)
            + ld * wld_ref[l]
            + be1_ref[l]
        )


_K1_BLK = 400


def _k1_body(h_ref, x_ref, v_ref, lm_ref, mh_ref, mx_ref, wa_ref, wb_ref,
             wv_ref, bv_ref, hn_ref, gr_ref, gc_ref, xn_ref, vn_ref,
             sv_ref):
    hin = h_ref[...]
    mhv = mh_ref[...]
    mxv = mx_ref[...]
    xc = x_ref[...] - lm_ref[...]
    vv = v_ref[...]
    pad = jnp.zeros((_K1_BLK, _GW - _D - 3), _f32)
    for to in range(_T):
        acc = jnp.zeros((_K1_BLK, _D), _f32)
        for ti in range(_T):
            acc = acc + jnp.dot(hin[ti], mhv[ti, to],
                                preferred_element_type=_f32)
        hn_ref[to] = acc
        ax = jnp.zeros((_K1_BLK, 3), _f32)
        av = jnp.zeros((_K1_BLK, 3), _f32)
        for ti in range(_T):
            ax = ax + xc[ti] * mxv[ti, to, 0, 0] + vv[ti] * mxv[ti, to, 1, 0]
            av = av + xc[ti] * mxv[ti, to, 0, 1] + vv[ti] * mxv[ti, to, 1, 1]
        xo = ax + lm_ref[to]
        xn_ref[to] = xo
        vn_ref[to] = av
        gr_ref[to] = jnp.concatenate(
            [jnp.dot(acc, wa_ref[...], preferred_element_type=_f32), xo, pad],
            axis=1)
        gc_ref[to] = jnp.concatenate(
            [jnp.dot(acc, wb_ref[...], preferred_element_type=_f32), xo, pad],
            axis=1)
        sv_ref[to] = (jnp.dot(acc, wv_ref[...], preferred_element_type=_f32)
                      + bv_ref[...])


_EB = 2000


def _make_edge_body(last):
    def body(gr_ref, gc_ref, b_ref, wr_ref, w2_ref, b2_ref,
             x1_ref, bx1_ref, x2_ref, out_ref):
        grv = gr_ref[...]
        gcv = gc_ref[...]
        a = grv[:, 0:_D] + gcv[:, 0:_D]
        d = grv[:, _D:_D + 3] - gcv[:, _D:_D + 3]
        radial = jnp.sum(d * d, axis=1, keepdims=True)
        pre = a + radial * wr_ref[...] + b_ref[...]
        pre = pre * jax.nn.sigmoid(pre)
        m = jnp.dot(pre, w2_ref[...], preferred_element_type=_f32) + b2_ref[...]
        m = m * jax.nn.sigmoid(m)
        if last:
            t1 = (jnp.dot(m, x1_ref[...], preferred_element_type=_f32)
                  + bx1_ref[...])
            t1 = t1 * jax.nn.sigmoid(t1)
            tx = jnp.dot(t1, x2_ref[...], preferred_element_type=_f32)
            out_ref[...] = jnp.concatenate(
                [d * tx, jnp.ones((_EB, 1), _f32), jnp.zeros((_EB, 12), _f32)],
                axis=1)
        else:
            out_ref[...] = m
    return body


def _hupd_body(p_ref, hn_ref, wa_ref, wb_ref, b1_ref, w2_ref, b2_ref, out_ref):
    magg = p_ref[0] + p_ref[1]
    hv = hn_ref[...]
    u = (jnp.dot(hv, wa_ref[...], preferred_element_type=_f32)
         + jnp.dot(magg, wb_ref[...], preferred_element_type=_f32)
         + b1_ref[...])
    u = u * jax.nn.sigmoid(u)
    out_ref[...] = (hv + jnp.dot(u, w2_ref[...], preferred_element_type=_f32)
                    + b2_ref[...])


def _xupd_body(p_ref, xn_ref, vn_ref, sv_ref, out_ref):
    acc = p_ref[0] + p_ref[1]
    cnt = jnp.maximum(acc[:, 3:4], 1.0)
    agg = acc[:, 0:3] / cnt
    vout = sv_ref[...] * vn_ref[...] + agg
    out_ref[...] = xn_ref[...] + vout


_CHG = 320


def _make_gather_edge():
    total = _ET // _CHG
    cpt = _BE // _CHG
    npair = -(-total // (2 * _NW))
    out_h = jax.ShapeDtypeStruct((_ET, _GW), _f32)
    mesh = plsc.VectorSubcoreMesh(core_axis_name="c", subcore_axis_name="s")

    @functools.partial(
        pl.kernel, mesh=mesh,
        out_type=(out_h, out_h),
        compiler_params=pltpu.CompilerParams(use_tc_tiling_on_sc=False),
        scratch_types=(
            [pltpu.VMEM((_CHG,), jnp.int32) for _ in range(4)]
            + [pltpu.VMEM((_CHG, _GW), _f32) for _ in range(4)]
            + [pltpu.SemaphoreType.DMA for _ in range(4)]
        ),
    )
    def k(tab_hr, tab_hc, idx_r, idx_c, o_hr, o_hc,
          iv0, iv1, iv2, iv3, r0, r1, r2, r3, s0, s1, s2, s3):
        idx_vs = (iv0, iv1, iv2, iv3)
        rows = (r0, r1, r2, r3)
        sems = (s0, s1, s2, s3)
        wid = lax.axis_index("s") * 2 + lax.axis_index("c")

        def body(kk, carry):
            gs = [wid + _NW * (2 * kk), wid + _NW * (2 * kk + 1)]
            for u, g in enumerate(gs):
                for s, (tab, idx) in enumerate(
                        ((tab_hr, idx_r), (tab_hc, idx_c))):
                    b = 2 * u + s
                    @pl.when(g < total)
                    def _(g=g, b=b, tab=tab, idx=idx):
                        t = g // cpt
                        j = g - t * cpt
                        off = t * _N
                        iv = idx_vs[b]
                        pltpu.sync_copy(idx.at[pl.ds(j * _CHG, _CHG)], iv)
                        for i in range(_CHG // 16):
                            iv[pl.ds(i * 16, 16)] = iv[pl.ds(i * 16, 16)] + off
                        pltpu.async_copy(tab.at[iv], rows[b], sems[b])
            for u, g in enumerate(gs):
                for s, o_h in enumerate((o_hr, o_hc)):
                    b = 2 * u + s
                    @pl.when(g < total)
                    def _(g=g, b=b, o_h=o_h):
                        pltpu.make_async_copy(
                            tab_hr.at[idx_vs[b]], rows[b], sems[b]).wait()
                        pltpu.sync_copy(rows[b], o_h.at[pl.ds(g * _CHG, _CHG)])
            return carry

        lax.fori_loop(0, npair, body, 0)

    return k


def _make_gather(n_tab, width, n_idx, t_rep, n_off):
    total = (n_idx * t_rep) // _CH
    cpt = n_idx // _CH
    niter = -(-total // _NW)
    out_sh = jax.ShapeDtypeStruct((n_idx * t_rep, width), _f32)
    mesh = plsc.VectorSubcoreMesh(core_axis_name="c", subcore_axis_name="s")

    @functools.partial(
        pl.kernel, mesh=mesh,
        out_type=(out_sh, out_sh),
        compiler_params=pltpu.CompilerParams(use_tc_tiling_on_sc=False),
        scratch_types=[
            pltpu.VMEM((_CH,), jnp.int32),
            pltpu.VMEM((_CH, width), _f32),
            pltpu.SemaphoreType.DMA,
        ],
    )
    def k(tab_r, tab_c, idx_r, idx_c, out_r, out_c, idx_v, row_v, sem):
        wid = lax.axis_index("s") * 2 + lax.axis_index("c")

        def run_side(tab, idx, out):
            def body(kk, carry):
                g = wid + _NW * kk

                @pl.when(g < total)
                def _():
                    t = g // cpt
                    j = g - t * cpt
                    pltpu.sync_copy(idx.at[pl.ds(j * _CH, _CH)], idx_v)
                    if t_rep > 1:
                        off = t * n_off
                        for i in range(_CH // 16):
                            idx_v[pl.ds(i * 16, 16)] = (
                                idx_v[pl.ds(i * 16, 16)] + off)
                    pltpu.async_copy(tab.at[idx_v], row_v, sem).wait()
                    pltpu.sync_copy(row_v, out.at[pl.ds(g * _CH, _CH)])
                return carry

            lax.fori_loop(0, niter, body, 0)

        run_side(tab_r, idx_r, out_r)
        run_side(tab_c, idx_c, out_c)

    return k


_SEG = 9216
_SROWS = _SEG // 16


def _make_scatter(width):
    total = _ET // _CH
    cpt = _BE // _CH
    niter = -(-total // _NW)
    rows = _SROWS
    mesh = plsc.VectorSubcoreMesh(core_axis_name="c", subcore_axis_name="s")

    @functools.partial(
        pl.kernel, mesh=mesh,
        out_type=jax.ShapeDtypeStruct((2 * _SEG, width), _f32),
        compiler_params=pltpu.CompilerParams(use_tc_tiling_on_sc=False),
        scratch_types=[
            pltpu.VMEM((_CH,), jnp.int32),
            pltpu.VMEM((_CH, width), _f32),
            pltpu.VMEM((rows, width), _f32),
            pltpu.VMEM_SHARED((_SEG, width), _f32),
        ],
    )
    def k(val_hbm, idx_hbm, zero_hbm, out_hbm, idx_v, val_v, blk_v, shared):
        cid = lax.axis_index("c")
        sid = lax.axis_index("s")
        wid = sid * 2 + cid
        pltpu.sync_copy(zero_hbm, shared.at[pl.ds(sid * rows, rows)])
        plsc.subcore_barrier()

        def body(kk, carry):
            g = wid + _NW * kk

            @pl.when(g < total)
            def _():
                t = g // cpt
                j = g - t * cpt
                pltpu.sync_copy(idx_hbm.at[pl.ds(j * _CH, _CH)], idx_v)
                off = t * _N
                for i in range(_CH // 16):
                    idx_v[pl.ds(i * 16, 16)] = idx_v[pl.ds(i * 16, 16)] + off
                pltpu.sync_copy(val_hbm.at[pl.ds(g * _CH, _CH)], val_v)
                pltpu.sync_copy(val_v, shared.at[idx_v], add=True)
            return carry

        lax.fori_loop(0, niter, body, 0)
        plsc.subcore_barrier()
        pltpu.sync_copy(shared.at[pl.ds(sid * rows, rows)], blk_v)
        pltpu.sync_copy(blk_v, out_hbm.at[pl.ds(cid * _SEG + sid * rows, rows)])

    return k


_make_gather = functools.lru_cache(maxsize=None)(_make_gather)
_make_gather_edge = functools.lru_cache(maxsize=None)(_make_gather_edge)
_make_scatter = functools.lru_cache(maxsize=None)(_make_scatter)


def _gather_loc(*args):
    return _make_gather(_BN, 16, _BE, 1, 0)(*args)


def _gather_edge(*args):
    return _make_gather_edge()(*args)


def _scatter_m(*args):
    return _make_scatter(_D)(*args)


def _scatter_x(*args):
    return _make_scatter(16)(*args)


def _timestep_embedding(num_timesteps, dim):
    half = dim // 2
    emb = math.log(10000.0) / (half - 1)
    freqs = jnp.exp(jnp.arange(half, dtype=_f32) * -emb)
    te = jnp.arange(num_timesteps, dtype=_f32)[:, None] * freqs[None, :]
    return jnp.concatenate([jnp.sin(te), jnp.cos(te)], axis=-1)


def _fold_time_conv(wr, wi):
    k = jnp.arange(_MODES, dtype=_f32)
    t = jnp.arange(_T, dtype=_f32)
    w_k = jnp.where(k == 0, 1.0, 2.0)
    ang = (2.0 * math.pi / _T) * k[None, None, :] * (
        t[None, :, None] - t[:, None, None])
    m = (jnp.einsum('itk,k,kco->itco', jnp.cos(ang), w_k, wr)
         - jnp.einsum('itk,k,kco->itco', jnp.sin(ang), w_k, wi)) / _T
    eye = jnp.eye(wr.shape[1], dtype=_f32)
    return m + jnp.eye(_T, dtype=_f32)[:, :, None, None] * eye[None, None]


def kernel(x_0, v_0, concatenated_features, edge_attr, source_node_indices,
           target_node_indices, params):
    P = params['layers']

    te = _timestep_embedding(_T, _TIME_EMB)
    time_emb = jnp.reshape(
        jnp.broadcast_to(te[:, None, :], (_T, _BN, _TIME_EMB)),
        (_BTN, _TIME_EMB))
    h2 = jnp.reshape(concatenated_features[..., -2:], (_BTN, 2))
    feat = jnp.concatenate([h2, time_emb], axis=1)
    x = jnp.reshape(x_0[..., :3], (_BTN, 3))
    v = jnp.reshape(v_0[..., :3], (_BTN, 3))
    loc = jnp.reshape(x_0[:, 0, :, :3], (_BN, 3))
    locpad = jnp.concatenate([loc, jnp.zeros((_BN, 13), _f32)], axis=1)
    rows0 = jnp.reshape(source_node_indices, (_BE,)).astype(jnp.int32)
    cols0 = jnp.reshape(target_node_indices, (_BE,)).astype(jnp.int32)
    ea = jnp.reshape(edge_attr, (_BE, 2))

    mh = [_fold_time_conv(P[l]['wt_r'], P[l]['wt_i']) for l in range(2)]
    mx = [_fold_time_conv(P[l]['wtx_r'], P[l]['wtx_i']) for l in range(2)]
    wea_s = jnp.stack([P[l]['We1'][129:131] for l in range(2)])
    wld_s = jnp.stack([P[l]['We1'][131] for l in range(2)])
    be1_s = jnp.stack([P[l]['be1'] for l in range(2)])
    z64 = jnp.zeros((_SROWS, _D), _f32)
    z16 = jnp.zeros((_SROWS, 16), _f32)

    x0p = jnp.reshape(jnp.transpose(x_0[..., :3], (2, 0, 1, 3)), (_N, 48))
    h, lm48 = pl.pallas_call(
        _prep_body,
        out_shape=(jax.ShapeDtypeStruct((_BTN, _D), _f32),
                   jax.ShapeDtypeStruct((1, 48), _f32)),
    )(feat, params['W_emb'], params['b_emb'][None, :], x0p)
    lm = jnp.reshape(
        jnp.broadcast_to(jnp.reshape(lm48, (_B * _T, 1, 3)), (_B * _T, _N, 3)),
        (_BTN, 3))

    gl_r, gl_c = _gather_loc(locpad, locpad, rows0, cols0)

    bias = pl.pallas_call(
        _bias_body,
        grid=(8,),
        in_specs=[
            pl.BlockSpec((_BE // 8, 16), lambda i: (i, 0)),
            pl.BlockSpec((_BE // 8, 16), lambda i: (i, 0)),
            pl.BlockSpec((_BE // 8, 2), lambda i: (i, 0)),
            pl.BlockSpec((2, 2, _D), lambda i: (0, 0, 0)),
            pl.BlockSpec((2, _D), lambda i: (0, 0)),
            pl.BlockSpec((2, _D), lambda i: (0, 0)),
        ],
        out_specs=pl.BlockSpec((2, _BE // 8, _D), lambda i: (0, i, 0)),
        out_shape=jax.ShapeDtypeStruct((2, _BE, _D), _f32),
    )(gl_r, gl_c, ea, wea_s, wld_s, be1_s)

    x_out = None
    for l in range(2):
        Pl = P[l]
        last = l == 1
        k1_out = pl.pallas_call(
            _k1_body,
            grid=(_BN // _K1_BLK,),
            in_specs=[
                pl.BlockSpec((_T, _K1_BLK, _D), lambda i: (0, i, 0)),
                pl.BlockSpec((_T, _K1_BLK, 3), lambda i: (0, i, 0)),
                pl.BlockSpec((_T, _K1_BLK, 3), lambda i: (0, i, 0)),
                pl.BlockSpec((_T, _K1_BLK, 3), lambda i: (0, i, 0)),
                pl.BlockSpec((_T, _T, _D, _D), lambda i: (0, 0, 0, 0)),
                pl.BlockSpec((_T, _T, 2, 2), lambda i: (0, 0, 0, 0)),
                pl.BlockSpec((_D, _D), lambda i: (0, 0)),
                pl.BlockSpec((_D, _D), lambda i: (0, 0)),
                pl.BlockSpec((_D, 1), lambda i: (0, 0)),
                pl.BlockSpec((1, 1), lambda i: (0, 0)),
            ],
            out_specs=[
                pl.BlockSpec((_T, _K1_BLK, _D), lambda i: (0, i, 0)),
                pl.BlockSpec((_T, _K1_BLK, _GW), lambda i: (0, i, 0)),
                pl.BlockSpec((_T, _K1_BLK, _GW), lambda i: (0, i, 0)),
                pl.BlockSpec((_T, _K1_BLK, 3), lambda i: (0, i, 0)),
                pl.BlockSpec((_T, _K1_BLK, 3), lambda i: (0, i, 0)),
                pl.BlockSpec((_T, _K1_BLK, 1), lambda i: (0, i, 0)),
            ],
            out_shape=(
                jax.ShapeDtypeStruct((_T, _BN, _D), _f32),
                jax.ShapeDtypeStruct((_T, _BN, _GW), _f32),
                jax.ShapeDtypeStruct((_T, _BN, _GW), _f32),
                jax.ShapeDtypeStruct((_T, _BN, 3), _f32),
                jax.ShapeDtypeStruct((_T, _BN, 3), _f32),
                jax.ShapeDtypeStruct((_T, _BN, 1), _f32),
            ),
        )(jnp.reshape(h, (_T, _BN, _D)), jnp.reshape(x, (_T, _BN, 3)),
          jnp.reshape(v, (_T, _BN, 3)), jnp.reshape(lm, (_T, _BN, 3)),
          mh[l], mx[l], Pl['We1'][0:_D], Pl['We1'][_D:2 * _D], Pl['Wv'],
          Pl['bv'][None, :])
        hn_t, gr_t, gc_t, xn_t, vn_t, sv_t = k1_out
        hn = jnp.reshape(hn_t, (_BTN, _D))
        tab_hr = jnp.reshape(gr_t, (_BTN, _GW))
        tab_hc = jnp.reshape(gc_t, (_BTN, _GW))
        xn = jnp.reshape(xn_t, (_BTN, 3))
        vn = jnp.reshape(vn_t, (_BTN, 3))

        g_hr, g_hc = _gather_edge(tab_hr, tab_hc, rows0, cols0)

        w_out = 16 if last else _D
        val = pl.pallas_call(
            _make_edge_body(last),
            grid=(_ET // _EB,),
            in_specs=[
                pl.BlockSpec((_EB, _GW), lambda i: (i, 0)),
                pl.BlockSpec((_EB, _GW), lambda i: (i, 0)),
                pl.BlockSpec((_EB, _D), lambda i: (i % (_BE // _EB), 0)),
                pl.BlockSpec((1, _D), lambda i: (0, 0)),
                pl.BlockSpec((_D, _D), lambda i: (0, 0)),
                pl.BlockSpec((1, _D), lambda i: (0, 0)),
                pl.BlockSpec((_D, _D), lambda i: (0, 0)),
                pl.BlockSpec((1, _D), lambda i: (0, 0)),
                pl.BlockSpec((_D, 1), lambda i: (0, 0)),
            ],
            out_specs=pl.BlockSpec((_EB, w_out), lambda i: (i, 0)),
            out_shape=jax.ShapeDtypeStruct((_ET, w_out), _f32),
        )(g_hr, g_hc, bias[l], Pl['We1'][128][None, :], Pl['We2'],
          Pl['be2'][None, :], Pl['Wx1'], Pl['bx1'][None, :], Pl['Wx2'])

        if not last:
            part = _scatter_m(val, rows0, z64)
            p = jnp.concatenate(
                [jnp.reshape(part, (2, _SEG, _D)),
                 jnp.zeros((2, _BTN - _SEG, _D), _f32)], axis=1)
            h = pl.pallas_call(
                _hupd_body,
                grid=(8,),
                in_specs=[
                    pl.BlockSpec((2, _BTN // 8, _D), lambda i: (0, i, 0)),
                    pl.BlockSpec((_BTN // 8, _D), lambda i: (i, 0)),
                    pl.BlockSpec((_D, _D), lambda i: (0, 0)),
                    pl.BlockSpec((_D, _D), lambda i: (0, 0)),
                    pl.BlockSpec((1, _D), lambda i: (0, 0)),
                    pl.BlockSpec((_D, _D), lambda i: (0, 0)),
                    pl.BlockSpec((1, _D), lambda i: (0, 0)),
                ],
                out_specs=pl.BlockSpec((_BTN // 8, _D), lambda i: (i, 0)),
                out_shape=jax.ShapeDtypeStruct((_BTN, _D), _f32),
            )(p, hn, Pl['Wh1'][0:_D], Pl['Wh1'][_D:2 * _D],
              Pl['bh1'][None, :], Pl['Wh2'], Pl['bh2'][None, :])
            x, v = xn, vn
        else:
            part = _scatter_x(val, rows0, z16)
            p = jnp.concatenate(
                [jnp.reshape(part, (2, _SEG, 16)),
                 jnp.zeros((2, _BTN - _SEG, 16), _f32)], axis=1)
            x_out = pl.pallas_call(
                _xupd_body,
                grid=(8,),
                in_specs=[
                    pl.BlockSpec((2, _BTN // 8, 16), lambda i: (0, i, 0)),
                    pl.BlockSpec((_BTN // 8, 3), lambda i: (i, 0)),
                    pl.BlockSpec((_BTN // 8, 3), lambda i: (i, 0)),
                    pl.BlockSpec((_BTN // 8, 1), lambda i: (i, 0)),
                ],
                out_specs=pl.BlockSpec((_BTN // 8, 3), lambda i: (i, 0)),
                out_shape=jax.ShapeDtypeStruct((_BTN, 3), _f32),
            )(p, xn, vn, jnp.reshape(sv_t, (_BTN, 1)))

    return jnp.reshape(x_out, (_B, _T, _N, 3))
